# Initial kernel scaffold; baseline (speedup 1.0000x reference)
import functools

import jax
import jax.numpy as jnp
from jax import lax
from jax.experimental import pallas as pl
from jax.experimental.pallas import tpu as pltpu
from jax.experimental.pallas import tpu_sc as plsc

N = 50000
D = 64
H = 32
E = 800000
NC, NS = 2, 16
N2 = 50176
E2 = 802816
STRIPE = N2 // NS
BATCH = 128
EPT = E2 // NS
NB = EPT // BATCH
EPT_A = E2 // (NC * NS)
NB_A = EPT_A // BATCH
PR = 448

_mesh = plsc.VectorSubcoreMesh(
    core_axis_name="c", subcore_axis_name="s", num_cores=NC, num_subcores=NS
)

_f32 = jnp.float32
_i32 = jnp.int32


def _splat(vec_ref, i):
    return plsc.load_gather(vec_ref, [jnp.zeros((16,), _i32) + i])


@functools.partial(
    pl.kernel,
    out_type=jax.ShapeDtypeStruct((4 * N2,), _f32),
    mesh=_mesh,
    scratch_types=[
        pltpu.VMEM((BATCH,), _i32),
        pltpu.VMEM((BATCH,), _i32),
        pltpu.VMEM((BATCH,), _f32),
        pltpu.VMEM((STRIPE,), _f32),
        pltpu.VMEM_SHARED((N2,), _f32),
        pltpu.VMEM_SHARED((N2,), _f32),
    ],
)
def _sums_kernel(ridx_hbm, cidx_hbm, ev_hbm, sums_hbm,
                 ridx_v, cidx_v, ev_v, stripe_v, rsum_sh, csum_sh):
    c = lax.axis_index("c")
    s = lax.axis_index("s")

    @pl.loop(0, STRIPE // 16)
    def _(i):
        stripe_v[pl.ds(i * 16, 16)] = jnp.zeros((16,), _f32)

    pltpu.sync_copy(stripe_v, rsum_sh.at[pl.ds(s * STRIPE, STRIPE)])
    pltpu.sync_copy(stripe_v, csum_sh.at[pl.ds(s * STRIPE, STRIPE)])
    plsc.subcore_barrier()

    base = (c * NS + s) * EPT_A

    @pl.loop(0, NB_A)
    def _(b):
        off = base + b * BATCH
        pltpu.sync_copy(ridx_hbm.at[pl.ds(off, BATCH)], ridx_v)
        pltpu.sync_copy(cidx_hbm.at[pl.ds(off, BATCH)], cidx_v)
        pltpu.sync_copy(ev_hbm.at[pl.ds(off, BATCH)], ev_v)
        pltpu.sync_copy(ev_v, rsum_sh.at[ridx_v], add=True)
        pltpu.sync_copy(ev_v, csum_sh.at[cidx_v], add=True)

    plsc.subcore_barrier()
    pltpu.sync_copy(rsum_sh.at[pl.ds(s * STRIPE, STRIPE)], stripe_v)
    pltpu.sync_copy(stripe_v, sums_hbm.at[pl.ds((c * 2 + 0) * N2 + s * STRIPE, STRIPE)])
    pltpu.sync_copy(csum_sh.at[pl.ds(s * STRIPE, STRIPE)], stripe_v)
    pltpu.sync_copy(stripe_v, sums_hbm.at[pl.ds((c * 2 + 1) * N2 + s * STRIPE, STRIPE)])


def _make_layer_kernel(final):
    n_out = 1 if final else 2
    out_type = [jax.ShapeDtypeStruct((2 * N2, H), _f32)] * n_out
    scratch = [
        pltpu.VMEM((BATCH,), _i32),
        pltpu.VMEM((BATCH,), _i32),
        pltpu.VMEM((BATCH,), _f32),
        pltpu.VMEM((BATCH, H), _f32),
        pltpu.VMEM((196, H), _f32),
        pltpu.VMEM((PR, H), _f32),
        pltpu.VMEM((PR, H), _f32),
        pltpu.VMEM((PR, H), _f32),
        pltpu.VMEM((PR,), _f32),
        pltpu.VMEM((PR,), _f32),
        pltpu.VMEM_SHARED((N2, H), _f32),
    ]

    def body(curc_hbm, ridx_hbm, cidx2_hbm, ev_hbm, rs_hbm, aux1_hbm, aux2_hbm,
             *refs):
        outs = refs[:n_out]
        (ridx_v, cidx_v, ev_v, rows_v, zbuf, sbuf, b1, b2, rs_v, cs_v,
         acc_sh) = refs[n_out:]
        h = lax.axis_index("c")
        s = lax.axis_index("s")

        @pl.loop(0, 196)
        def _(r):
            zbuf[r, pl.ds(0, 16)] = jnp.zeros((16,), _f32)
            zbuf[r, pl.ds(16, 16)] = jnp.zeros((16,), _f32)

        @pl.loop(0, STRIPE // 196)
        def _(i):
            pltpu.sync_copy(zbuf, acc_sh.at[pl.ds(s * STRIPE + i * 196, 196)])

        plsc.subcore_barrier()

        base = s * EPT

        @pl.loop(0, NB)
        def _(b):
            off = base + b * BATCH
            pltpu.sync_copy(ridx_hbm.at[pl.ds(off, BATCH)], ridx_v)
            pltpu.sync_copy(cidx2_hbm.at[pl.ds(h * E2 + off, BATCH)], cidx_v)
            pltpu.sync_copy(ev_hbm.at[pl.ds(off, BATCH)], ev_v)
            pltpu.sync_copy(curc_hbm.at[cidx_v], rows_v)

            @pl.loop(0, BATCH // 16)
            def _(g):
                for j in range(16):
                    e = g * 16 + j
                    w = _splat(ev_v, e)
                    rows_v[e, pl.ds(0, 16)] = rows_v[e, pl.ds(0, 16)] * w
                    rows_v[e, pl.ds(16, 16)] = rows_v[e, pl.ds(16, 16)] * w

            pltpu.sync_copy(rows_v, acc_sh.at[ridx_v], add=True)

        plsc.subcore_barrier()

        @pl.loop(0, STRIPE // PR)
        def _(p):
            r0 = s * STRIPE + p * PR
            pltpu.sync_copy(acc_sh.at[pl.ds(r0, PR)], sbuf)
            pltpu.sync_copy(rs_hbm.at[pl.ds(r0, PR)], rs_v)
            if final:
                pltpu.sync_copy(aux1_hbm.at[pl.ds(h * N2 + r0, PR)], b1)
                pltpu.sync_copy(aux2_hbm.at[pl.ds(h * N2 + r0, PR)], b2)
            else:
                pltpu.sync_copy(aux1_hbm.at[pl.ds(r0, PR)], cs_v)

            @pl.loop(0, PR // 16)
            def _(g):
                for j in range(16):
                    n = g * 16 + j
                    rb = _splat(rs_v, n)
                    if not final:
                        cb = _splat(cs_v, n)
                    for half in (0, 16):
                        x = sbuf[n, pl.ds(half, 16)] * rb
                        cur = jnp.maximum(x, x * 0.01)
                        if final:
                            sbuf[n, pl.ds(half, 16)] = (
                                b1[n, pl.ds(half, 16)] + b2[n, pl.ds(half, 16)] + cur
                            ) * (1.0 / 3.0)
                        else:
                            b1[n, pl.ds(half, 16)] = cur
                            b2[n, pl.ds(half, 16)] = cur * cb

            if final:
                pltpu.sync_copy(sbuf, outs[0].at[pl.ds(h * N2 + r0, PR)])
            else:
                pltpu.sync_copy(b1, outs[0].at[pl.ds(h * N2 + r0, PR)])
                pltpu.sync_copy(b2, outs[1].at[pl.ds(h * N2 + r0, PR)])

    return pl.kernel(body, out_type=out_type, mesh=_mesh, scratch_types=scratch)


_layer_kernel = _make_layer_kernel(final=False)
_layer_final_kernel = _make_layer_kernel(final=True)


def _norm_body(sums_ref, rs_ref, cs_ref):
    row = sums_ref[0] + sums_ref[2]
    col = sums_ref[1] + sums_ref[3]
    rs_ref[...] = 1.0 / (jnp.sqrt(row) + 1e-8)
    cs_ref[...] = 1.0 / (jnp.sqrt(col) + 1e-8)


_norm_call = pl.pallas_call(
    _norm_body,
    grid=(N2 // 128 // 8,),
    in_specs=[pl.BlockSpec((4, 8, 128), lambda i: (0, i, 0))],
    out_specs=[
        pl.BlockSpec((8, 128), lambda i: (i, 0)),
        pl.BlockSpec((8, 128), lambda i: (i, 0)),
    ],
    out_shape=[jax.ShapeDtypeStruct((N2 // 128, 128), _f32)] * 2,
)


def _prescale_body(f_ref, cs_ref, curc_ref, fs_ref):
    c = cs_ref[...]
    f = f_ref[...]
    lo = f[:, :H]
    hi = f[:, H:]
    fs_ref[0] = lo
    fs_ref[1] = hi
    curc_ref[0] = lo * c
    curc_ref[1] = hi * c


_prescale_call = pl.pallas_call(
    _prescale_body,
    grid=(N2 // 448,),
    in_specs=[
        pl.BlockSpec((448, D), lambda i: (i, 0)),
        pl.BlockSpec((448, 1), lambda i: (i, 0)),
    ],
    out_specs=[
        pl.BlockSpec((2, 448, H), lambda i: (0, i, 0)),
        pl.BlockSpec((2, 448, H), lambda i: (0, i, 0)),
    ],
    out_shape=[jax.ShapeDtypeStruct((2, N2, H), _f32)] * 2,
)


@jax.jit
def _impl(users_feature, groups_feature, items_feature, edge_vals, edge_index):
    pad = E2 - E
    row = edge_index[0].astype(_i32)
    col = edge_index[1].astype(_i32)
    extra = (jnp.arange(pad, dtype=_i32) * 37) % N
    ridx_p = jnp.concatenate([row, extra])
    cidx_p = jnp.concatenate([col, extra])
    ev_p = jnp.concatenate([edge_vals, jnp.zeros((pad,), _f32)])
    cidx2 = jnp.concatenate([cidx_p, cidx_p + N2])

    sums4 = _sums_kernel(ridx_p, cidx_p, ev_p)
    rs2d, cs2d = _norm_call(sums4.reshape(4, N2 // 128, 128))
    rs = rs2d.reshape(N2)
    cs = cs2d.reshape(N2)

    feats = jnp.concatenate([users_feature, groups_feature, items_feature], axis=0)
    feats_p = jnp.pad(feats, ((0, N2 - N), (0, 0)))
    curc0, fsplit = _prescale_call(feats_p, cs.reshape(N2, 1))
    curc0 = curc0.reshape(2 * N2, H)
    fsplit = fsplit.reshape(2 * N2, H)

    cur1, curc1 = _layer_kernel(curc0, ridx_p, cidx2, ev_p, rs, cs, cs)
    outg = _layer_final_kernel(curc1, ridx_p, cidx2, ev_p, rs, fsplit, cur1)
    if isinstance(outg, (list, tuple)):
        outg = outg[0]
    return jnp.concatenate([outg[:N], outg[N2:N2 + N]], axis=1)


def kernel(users_feature, groups_feature, items_feature, edge_vals, edge_index):
    return _impl(users_feature, groups_feature, items_feature, edge_vals,
                 edge_index)


# speedup vs baseline: 5.4893x; 5.4893x over previous
import functools

import jax
import jax.numpy as jnp
from jax import lax
from jax.experimental import pallas as pl
from jax.experimental.pallas import tpu as pltpu
from jax.experimental.pallas import tpu_sc as plsc

N = 50000
D = 64
H = 32
E = 800000
NC, NS = 2, 16
N2 = 50176
E2 = 802816
STRIPE = N2 // NS
BATCH = 128
EPT = E2 // NS
NB = EPT // BATCH
EPT_A = E2 // (NC * NS)
NB_A = EPT_A // BATCH
PR = 448
F = 2 * N2 * H

_mesh = plsc.VectorSubcoreMesh(
    core_axis_name="c", subcore_axis_name="s", num_cores=NC, num_subcores=NS
)

_f32 = jnp.float32
_i32 = jnp.int32

_sc_params = pltpu.CompilerParams(needs_layout_passes=False,
                                  use_tc_tiling_on_sc=False)


def _splat(vec_ref, i):
    return plsc.load_gather(vec_ref, [jnp.zeros((16,), _i32) + i])


@functools.partial(
    pl.kernel,
    out_type=jax.ShapeDtypeStruct((4 * N2,), _f32),
    mesh=_mesh,
    scratch_types=[
        pltpu.VMEM((BATCH,), _i32),
        pltpu.VMEM((BATCH,), _i32),
        pltpu.VMEM((BATCH,), _f32),
        pltpu.VMEM((STRIPE,), _f32),
        pltpu.VMEM_SHARED((N2,), _f32),
        pltpu.VMEM_SHARED((N2,), _f32),
    ],
    compiler_params=_sc_params,
)
def _sums_kernel(ridx_hbm, cidx_hbm, ev_hbm, sums_hbm,
                 ridx_v, cidx_v, ev_v, stripe_v, rsum_sh, csum_sh):
    c = lax.axis_index("c")
    s = lax.axis_index("s")

    @pl.loop(0, STRIPE // 16)
    def _(i):
        stripe_v[pl.ds(i * 16, 16)] = jnp.zeros((16,), _f32)

    pltpu.sync_copy(stripe_v, rsum_sh.at[pl.ds(s * STRIPE, STRIPE)])
    pltpu.sync_copy(stripe_v, csum_sh.at[pl.ds(s * STRIPE, STRIPE)])
    plsc.subcore_barrier()

    base = (c * NS + s) * EPT_A

    @pl.loop(0, NB_A)
    def _(b):
        off = base + b * BATCH
        pltpu.sync_copy(ridx_hbm.at[pl.ds(off, BATCH)], ridx_v)
        pltpu.sync_copy(cidx_hbm.at[pl.ds(off, BATCH)], cidx_v)
        pltpu.sync_copy(ev_hbm.at[pl.ds(off, BATCH)], ev_v)
        pltpu.sync_copy(ev_v, rsum_sh.at[ridx_v], add=True)
        pltpu.sync_copy(ev_v, csum_sh.at[cidx_v], add=True)

    plsc.subcore_barrier()
    pltpu.sync_copy(rsum_sh.at[pl.ds(s * STRIPE, STRIPE)], stripe_v)
    pltpu.sync_copy(stripe_v, sums_hbm.at[pl.ds((c * 2 + 0) * N2 + s * STRIPE, STRIPE)])
    pltpu.sync_copy(csum_sh.at[pl.ds(s * STRIPE, STRIPE)], stripe_v)
    pltpu.sync_copy(stripe_v, sums_hbm.at[pl.ds((c * 2 + 1) * N2 + s * STRIPE, STRIPE)])


@functools.partial(
    pl.kernel,
    out_type=jax.ShapeDtypeStruct((E2,), _f32),
    mesh=_mesh,
    scratch_types=[
        pltpu.VMEM((N2,), _f32),
        pltpu.VMEM((N2,), _f32),
        pltpu.VMEM((BATCH,), _i32),
        pltpu.VMEM((BATCH,), _i32),
        pltpu.VMEM((BATCH,), _f32),
        pltpu.VMEM((BATCH,), _f32),
    ],
    compiler_params=_sc_params,
)
def _weights_kernel(ridx_hbm, cidx_hbm, ev_hbm, rs_hbm, cs_hbm, nv_hbm,
                    rs_t, cs_t, ridx_v, cidx_v, ev_v, nv_v):
    c = lax.axis_index("c")
    s = lax.axis_index("s")
    pltpu.sync_copy(rs_hbm, rs_t)
    pltpu.sync_copy(cs_hbm, cs_t)
    base = (c * NS + s) * EPT_A

    @pl.loop(0, NB_A)
    def _(b):
        off = base + b * BATCH
        pltpu.sync_copy(ridx_hbm.at[pl.ds(off, BATCH)], ridx_v)
        pltpu.sync_copy(cidx_hbm.at[pl.ds(off, BATCH)], cidx_v)
        pltpu.sync_copy(ev_hbm.at[pl.ds(off, BATCH)], ev_v)

        @pl.loop(0, BATCH // 16)
        def _(g):
            sl = pl.ds(g * 16, 16)
            nv_v[sl] = (ev_v[sl]
                        * plsc.load_gather(cs_t, [cidx_v[sl]])
                        * plsc.load_gather(rs_t, [ridx_v[sl]]))

        pltpu.sync_copy(nv_v, nv_hbm.at[pl.ds(off, BATCH)])


@functools.partial(
    pl.kernel,
    out_type=jax.ShapeDtypeStruct((2 * N2, H), _f32),
    mesh=_mesh,
    scratch_types=[
        pltpu.VMEM((BATCH,), _i32),
        pltpu.VMEM((BATCH,), _i32),
        pltpu.VMEM((BATCH,), _i32),
        pltpu.VMEM((BATCH,), _f32),
        pltpu.VMEM((BATCH, H), _f32),
        pltpu.VMEM((PR, H), _f32),
        pltpu.VMEM_SHARED((N2, H), _f32),
    ],
    compiler_params=_sc_params,
)
def _layer_kernel(curg_hbm, ridx_hbm, cidx_hbm, nv_hbm,
                  out_hbm, ridx_v, cidx_v, tidx_v, nv_v,
                  rows_v, sbuf, acc_sh):
    h = lax.axis_index("c")
    s = lax.axis_index("s")
    curg2d = curg_hbm

    @pl.loop(0, PR)
    def _(r):
        sbuf[r, pl.ds(0, 16)] = jnp.zeros((16,), _f32)
        sbuf[r, pl.ds(16, 16)] = jnp.zeros((16,), _f32)

    @pl.loop(0, STRIPE // PR)
    def _(i):
        pltpu.sync_copy(sbuf, acc_sh.at[pl.ds(s * STRIPE + i * PR, PR)])

    plsc.subcore_barrier()

    base = s * EPT
    toff = h * N2

    @pl.loop(0, NB)
    def _(b):
        off = base + b * BATCH
        pltpu.sync_copy(ridx_hbm.at[pl.ds(off, BATCH)], ridx_v)
        pltpu.sync_copy(cidx_hbm.at[pl.ds(off, BATCH)], cidx_v)
        pltpu.sync_copy(nv_hbm.at[pl.ds(off, BATCH)], nv_v)

        @pl.loop(0, BATCH // 16)
        def _(g):
            sl = pl.ds(g * 16, 16)
            tidx_v[sl] = cidx_v[sl] + toff

        pltpu.sync_copy(curg2d.at[tidx_v], rows_v)

        @pl.loop(0, BATCH // 16)
        def _(g):
            for j in range(16):
                e = g * 16 + j
                w = _splat(nv_v, e)
                rows_v[e, pl.ds(0, 16)] = rows_v[e, pl.ds(0, 16)] * w
                rows_v[e, pl.ds(16, 16)] = rows_v[e, pl.ds(16, 16)] * w

        pltpu.sync_copy(rows_v, acc_sh.at[ridx_v], add=True)

    plsc.subcore_barrier()

    @pl.loop(0, STRIPE // PR)
    def _(p):
        r0 = s * STRIPE + p * PR
        pltpu.sync_copy(acc_sh.at[pl.ds(r0, PR)], sbuf)

        @pl.loop(0, PR)
        def _(n):
            for half in (0, 16):
                x = sbuf[n, pl.ds(half, 16)]
                sbuf[n, pl.ds(half, 16)] = jnp.maximum(x, x * 0.01)

        pltpu.sync_copy(sbuf, out_hbm.at[pl.ds(h * N2 + r0, PR)])


def _norm_body(sums_ref, rs_ref, cs_ref):
    row = sums_ref[0] + sums_ref[2]
    col = sums_ref[1] + sums_ref[3]
    rs_ref[...] = 1.0 / (jnp.sqrt(row) + 1e-8)
    cs_ref[...] = 1.0 / (jnp.sqrt(col) + 1e-8)


_norm_call = pl.pallas_call(
    _norm_body,
    grid=(N2 // 128 // 8,),
    in_specs=[pl.BlockSpec((4, 8, 128), lambda i: (0, i, 0))],
    out_specs=[
        pl.BlockSpec((8, 128), lambda i: (i, 0)),
        pl.BlockSpec((8, 128), lambda i: (i, 0)),
    ],
    out_shape=[jax.ShapeDtypeStruct((N2 // 128, 128), _f32)] * 2,
)

_CB = 1568


def _combine_body(f_ref, c1_ref, c2_ref, o_ref):
    o_ref[...] = (f_ref[...] + c1_ref[...] + c2_ref[...]) * (1.0 / 3.0)


_combine_call = pl.pallas_call(
    _combine_body,
    grid=(2 * N2 // _CB,),
    in_specs=[pl.BlockSpec((_CB, H), lambda i: (i, 0))] * 3,
    out_specs=pl.BlockSpec((_CB, H), lambda i: (i, 0)),
    out_shape=jax.ShapeDtypeStruct((2 * N2, H), _f32),
)


@jax.jit
def _impl(users_feature, groups_feature, items_feature, edge_vals, edge_index):
    pad = E2 - E
    row = edge_index[0].astype(_i32)
    col = edge_index[1].astype(_i32)
    extra = (jnp.arange(pad, dtype=_i32) * 37) % N
    ridx_p = jnp.concatenate([row, extra])
    cidx_p = jnp.concatenate([col, extra])
    ev_p = jnp.concatenate([edge_vals, jnp.zeros((pad,), _f32)])

    sums4 = _sums_kernel(ridx_p, cidx_p, ev_p)
    rs2d, cs2d = _norm_call(sums4.reshape(4, N2 // 128, 128))
    rs = rs2d.reshape(N2)
    cs = cs2d.reshape(N2)

    nv = _weights_kernel(ridx_p, cidx_p, ev_p, rs, cs)

    feats = jnp.concatenate([users_feature, groups_feature, items_feature], axis=0)
    feats_p = jnp.pad(feats, ((0, N2 - N), (0, 0)))
    fg = feats_p.reshape(N2, 2, H).transpose(1, 0, 2).reshape(2 * N2, H)

    def _step(cur, _):
        nxt = _layer_kernel(cur, ridx_p, cidx_p, nv)
        return nxt, nxt

    _, curs = lax.scan(_step, fg, None, length=2)
    cur1, cur2 = curs[0], curs[1]

    outf = _combine_call(fg, cur1, cur2)
    outg = outf.reshape(2, N2, H)
    return jnp.concatenate([outg[0, :N], outg[1, :N]], axis=1)


def kernel(users_feature, groups_feature, items_feature, edge_vals, edge_index):
    return _impl(users_feature, groups_feature, items_feature, edge_vals,
                 edge_index)


# speedup vs baseline: 8.7818x; 1.5998x over previous
import functools

import jax
import jax.numpy as jnp
from jax import lax
from jax.experimental import pallas as pl
from jax.experimental.pallas import tpu as pltpu
from jax.experimental.pallas import tpu_sc as plsc

N = 50000
D = 64
H = 32
E = 800000
NC, NS = 2, 16
N2 = 50176
E2 = 802816
STRIPE = N2 // NS
BATCH = 128
EPT = E2 // NS
NB = EPT // BATCH
NBT = E2 // BATCH
EPT_A = E2 // (NC * NS)
NB_A = EPT_A // BATCH
K = 8
NCH = NB // K
PR = 224
F = 2 * N2 * H

_mesh = plsc.VectorSubcoreMesh(
    core_axis_name="c", subcore_axis_name="s", num_cores=NC, num_subcores=NS
)

_f32 = jnp.float32
_i32 = jnp.int32

_sc_params = pltpu.CompilerParams(needs_layout_passes=False,
                                  use_tc_tiling_on_sc=False)


def _splat(vec_ref, i):
    return plsc.load_gather(vec_ref, [jnp.zeros((16,), _i32) + i])


@functools.partial(
    pl.kernel,
    out_type=jax.ShapeDtypeStruct((4 * N2,), _f32),
    mesh=_mesh,
    scratch_types=[
        pltpu.VMEM((BATCH,), _i32),
        pltpu.VMEM((BATCH,), _i32),
        pltpu.VMEM((BATCH,), _f32),
        pltpu.VMEM((STRIPE,), _f32),
        pltpu.VMEM_SHARED((N2,), _f32),
        pltpu.VMEM_SHARED((N2,), _f32),
    ],
    compiler_params=_sc_params,
)
def _sums_kernel(ridx_hbm, cidx_hbm, ev_hbm, sums_hbm,
                 ridx_v, cidx_v, ev_v, stripe_v, rsum_sh, csum_sh):
    c = lax.axis_index("c")
    s = lax.axis_index("s")

    @pl.loop(0, STRIPE // 16)
    def _(i):
        stripe_v[pl.ds(i * 16, 16)] = jnp.zeros((16,), _f32)

    pltpu.sync_copy(stripe_v, rsum_sh.at[pl.ds(s * STRIPE, STRIPE)])
    pltpu.sync_copy(stripe_v, csum_sh.at[pl.ds(s * STRIPE, STRIPE)])
    plsc.subcore_barrier()

    base = (c * NS + s) * EPT_A

    @pl.loop(0, NB_A)
    def _(b):
        off = base + b * BATCH
        pltpu.sync_copy(ridx_hbm.at[pl.ds(off, BATCH)], ridx_v)
        pltpu.sync_copy(cidx_hbm.at[pl.ds(off, BATCH)], cidx_v)
        pltpu.sync_copy(ev_hbm.at[pl.ds(off, BATCH)], ev_v)
        pltpu.sync_copy(ev_v, rsum_sh.at[ridx_v], add=True)
        pltpu.sync_copy(ev_v, csum_sh.at[cidx_v], add=True)

    plsc.subcore_barrier()
    pltpu.sync_copy(rsum_sh.at[pl.ds(s * STRIPE, STRIPE)], stripe_v)
    pltpu.sync_copy(stripe_v, sums_hbm.at[pl.ds((c * 2 + 0) * N2 + s * STRIPE, STRIPE)])
    pltpu.sync_copy(csum_sh.at[pl.ds(s * STRIPE, STRIPE)], stripe_v)
    pltpu.sync_copy(stripe_v, sums_hbm.at[pl.ds((c * 2 + 1) * N2 + s * STRIPE, STRIPE)])


@functools.partial(
    pl.kernel,
    out_type=jax.ShapeDtypeStruct((E2,), _f32),
    mesh=_mesh,
    scratch_types=[
        pltpu.VMEM((N2,), _f32),
        pltpu.VMEM((N2,), _f32),
        pltpu.VMEM((BATCH,), _i32),
        pltpu.VMEM((BATCH,), _i32),
        pltpu.VMEM((BATCH,), _f32),
        pltpu.VMEM((BATCH,), _f32),
    ],
    compiler_params=_sc_params,
)
def _weights_kernel(ridx_hbm, cidx_hbm, ev_hbm, rs_hbm, cs_hbm, nv_hbm,
                    rs_t, cs_t, ridx_v, cidx_v, ev_v, nv_v):
    c = lax.axis_index("c")
    s = lax.axis_index("s")
    pltpu.sync_copy(rs_hbm, rs_t)
    pltpu.sync_copy(cs_hbm, cs_t)
    base = (c * NS + s) * EPT_A

    @pl.loop(0, NB_A)
    def _(b):
        off = base + b * BATCH
        pltpu.sync_copy(ridx_hbm.at[pl.ds(off, BATCH)], ridx_v)
        pltpu.sync_copy(cidx_hbm.at[pl.ds(off, BATCH)], cidx_v)
        pltpu.sync_copy(ev_hbm.at[pl.ds(off, BATCH)], ev_v)

        @pl.loop(0, BATCH // 16)
        def _(g):
            sl = pl.ds(g * 16, 16)
            nv_v[sl] = (ev_v[sl]
                        * plsc.load_gather(cs_t, [cidx_v[sl]])
                        * plsc.load_gather(rs_t, [ridx_v[sl]]))

        pltpu.sync_copy(nv_v, nv_hbm.at[pl.ds(off, BATCH)])


@functools.partial(
    pl.kernel,
    out_type=jax.ShapeDtypeStruct((2 * N2, H), _f32),
    mesh=_mesh,
    scratch_types=[
        pltpu.VMEM((K, BATCH), _i32),
        pltpu.VMEM((K, BATCH), _i32),
        pltpu.VMEM((K, BATCH), _f32),
        pltpu.VMEM((BATCH, H), _f32),
        pltpu.VMEM((BATCH, H), _f32),
        pltpu.VMEM((PR, H), _f32),
        pltpu.VMEM_SHARED((N2, H), _f32),
        pltpu.SemaphoreType.DMA,
        pltpu.SemaphoreType.DMA,
        pltpu.SemaphoreType.DMA,
        pltpu.SemaphoreType.DMA,
    ],
    compiler_params=_sc_params,
)
def _layer_kernel(curg_hbm, tidx2_hbm, ridx2_hbm, nv2_hbm,
                  out_hbm, tidx_c, ridx_c, nv_c,
                  rows0, rows1, sbuf, acc_sh,
                  gsem0, gsem1, ssem0, ssem1):
    h = lax.axis_index("c")
    s = lax.axis_index("s")
    rbuf = (rows0, rows1)
    gsem = (gsem0, gsem1)
    ssem = (ssem0, ssem1)

    @pl.loop(0, PR)
    def _(r):
        sbuf[r, pl.ds(0, 16)] = jnp.zeros((16,), _f32)
        sbuf[r, pl.ds(16, 16)] = jnp.zeros((16,), _f32)

    @pl.loop(0, STRIPE // PR)
    def _(i):
        pltpu.sync_copy(sbuf, acc_sh.at[pl.ds(s * STRIPE + i * PR, PR)])

    plsc.subcore_barrier()

    @pl.loop(0, NCH)
    def _(c):
        row0 = s * NB + c * K
        pltpu.sync_copy(tidx2_hbm.at[h, pl.ds(row0, K)], tidx_c)
        pltpu.sync_copy(ridx2_hbm.at[pl.ds(row0, K)], ridx_c)
        pltpu.sync_copy(nv2_hbm.at[pl.ds(row0, K)], nv_c)

        hg = [None] * K
        hs = [None] * K
        hg[0] = pltpu.async_copy(curg_hbm.at[tidx_c.at[0]], rbuf[0], gsem[0])
        for j in range(K):
            p = j & 1
            if j >= 1:
                hs[j - 1].wait()
            if j + 1 < K:
                hg[j + 1] = pltpu.async_copy(
                    curg_hbm.at[tidx_c.at[j + 1]], rbuf[1 - p], gsem[1 - p])
            hg[j].wait()

            @pl.loop(0, BATCH // 16)
            def _(g):
                for jj in range(16):
                    e = g * 16 + jj
                    w = plsc.load_gather(
                        nv_c, [jnp.zeros((16,), _i32) + j,
                               jnp.zeros((16,), _i32) + e])
                    rbuf[p][e, pl.ds(0, 16)] = rbuf[p][e, pl.ds(0, 16)] * w
                    rbuf[p][e, pl.ds(16, 16)] = rbuf[p][e, pl.ds(16, 16)] * w

            hs[j] = pltpu.async_copy(rbuf[p], acc_sh.at[ridx_c.at[j]],
                                     ssem[p], add=True)
        hs[K - 1].wait()

    plsc.subcore_barrier()

    @pl.loop(0, STRIPE // PR)
    def _(p):
        r0 = s * STRIPE + p * PR
        pltpu.sync_copy(acc_sh.at[pl.ds(r0, PR)], sbuf)

        @pl.loop(0, PR)
        def _(n):
            for half in (0, 16):
                x = sbuf[n, pl.ds(half, 16)]
                sbuf[n, pl.ds(half, 16)] = jnp.maximum(x, x * 0.01)

        pltpu.sync_copy(sbuf, out_hbm.at[pl.ds(h * N2 + r0, PR)])


def _norm_body(sums_ref, rs_ref, cs_ref):
    row = sums_ref[0] + sums_ref[2]
    col = sums_ref[1] + sums_ref[3]
    rs_ref[...] = 1.0 / (jnp.sqrt(row) + 1e-8)
    cs_ref[...] = 1.0 / (jnp.sqrt(col) + 1e-8)


_norm_call = pl.pallas_call(
    _norm_body,
    grid=(N2 // 128 // 8,),
    in_specs=[pl.BlockSpec((4, 8, 128), lambda i: (0, i, 0))],
    out_specs=[
        pl.BlockSpec((8, 128), lambda i: (i, 0)),
        pl.BlockSpec((8, 128), lambda i: (i, 0)),
    ],
    out_shape=[jax.ShapeDtypeStruct((N2 // 128, 128), _f32)] * 2,
)

_CB = 1568


def _combine_body(f_ref, c1_ref, c2_ref, o_ref):
    o_ref[...] = (f_ref[...] + c1_ref[...] + c2_ref[...]) * (1.0 / 3.0)


_combine_call = pl.pallas_call(
    _combine_body,
    grid=(2 * N2 // _CB,),
    in_specs=[pl.BlockSpec((_CB, H), lambda i: (i, 0))] * 3,
    out_specs=pl.BlockSpec((_CB, H), lambda i: (i, 0)),
    out_shape=jax.ShapeDtypeStruct((2 * N2, H), _f32),
)


@jax.jit
def _impl(users_feature, groups_feature, items_feature, edge_vals, edge_index):
    pad = E2 - E
    row = edge_index[0].astype(_i32)
    col = edge_index[1].astype(_i32)
    extra = (jnp.arange(pad, dtype=_i32) * 37) % N
    ridx_p = jnp.concatenate([row, extra])
    cidx_p = jnp.concatenate([col, extra])
    ev_p = jnp.concatenate([edge_vals, jnp.zeros((pad,), _f32)])

    sums4 = _sums_kernel(ridx_p, cidx_p, ev_p)
    rs2d, cs2d = _norm_call(sums4.reshape(4, N2 // 128, 128))
    rs = rs2d.reshape(N2)
    cs = cs2d.reshape(N2)

    nv = _weights_kernel(ridx_p, cidx_p, ev_p, rs, cs)

    ridx2 = ridx_p.reshape(NBT, BATCH)
    tidx2 = jnp.stack([cidx_p, cidx_p + N2]).reshape(2, NBT, BATCH)
    nv2 = nv.reshape(NBT, BATCH)

    feats = jnp.concatenate([users_feature, groups_feature, items_feature], axis=0)
    feats_p = jnp.pad(feats, ((0, N2 - N), (0, 0)))
    fg = feats_p.reshape(N2, 2, H).transpose(1, 0, 2).reshape(2 * N2, H)

    def _step(cur, _):
        nxt = _layer_kernel(cur, tidx2, ridx2, nv2)
        return nxt, nxt

    _, curs = lax.scan(_step, fg, None, length=2)
    cur1, cur2 = curs[0], curs[1]

    outf = _combine_call(fg, cur1, cur2)
    outg = outf.reshape(2, N2, H)
    return jnp.concatenate([outg[0, :N], outg[1, :N]], axis=1)


def kernel(users_feature, groups_feature, items_feature, edge_vals, edge_index):
    return _impl(users_feature, groups_feature, items_feature, edge_vals,
                 edge_index)


# speedup vs baseline: 11.2289x; 1.2787x over previous
import functools

import jax
import jax.numpy as jnp
from jax import lax
from jax.experimental import pallas as pl
from jax.experimental.pallas import tpu as pltpu
from jax.experimental.pallas import tpu_sc as plsc

N = 50000
D = 64
H = 32
E = 800000
NC, NS = 2, 16
N2 = 50176
E2 = 802816
STRIPE = N2 // NS
BATCH = 128
EPT = E2 // NS
NB = EPT // BATCH
NBT = E2 // BATCH
EPT_A = E2 // (NC * NS)
NB_A = EPT_A // BATCH
KA = 7
NCH_A = NB_A // KA
CW = 14 * BATCH
NCH_W = EPT_A // CW
K = 8
NCH = NB // K
PR = 224
F = 2 * N2 * H

_mesh = plsc.VectorSubcoreMesh(
    core_axis_name="c", subcore_axis_name="s", num_cores=NC, num_subcores=NS
)

_f32 = jnp.float32
_i32 = jnp.int32

_sc_params = pltpu.CompilerParams(needs_layout_passes=False,
                                  use_tc_tiling_on_sc=False)


def _splat(vec_ref, i):
    return plsc.load_gather(vec_ref, [jnp.zeros((16,), _i32) + i])


@functools.partial(
    pl.kernel,
    out_type=jax.ShapeDtypeStruct((4 * N2,), _f32),
    mesh=_mesh,
    scratch_types=[
        pltpu.VMEM((KA, BATCH), _i32),
        pltpu.VMEM((KA, BATCH), _i32),
        pltpu.VMEM((KA, BATCH), _f32),
        pltpu.VMEM((STRIPE,), _f32),
        pltpu.VMEM_SHARED((N2,), _f32),
        pltpu.VMEM_SHARED((N2,), _f32),
        pltpu.SemaphoreType.DMA,
    ],
    compiler_params=_sc_params,
)
def _sums_kernel(ridx2_hbm, cidx2_hbm, ev2_hbm, sums_hbm,
                 ridx_c, cidx_c, ev_c, stripe_v, rsum_sh, csum_sh, ssem):
    c = lax.axis_index("c")
    s = lax.axis_index("s")

    @pl.loop(0, STRIPE // 16)
    def _(i):
        stripe_v[pl.ds(i * 16, 16)] = jnp.zeros((16,), _f32)

    pltpu.sync_copy(stripe_v, rsum_sh.at[pl.ds(s * STRIPE, STRIPE)])
    pltpu.sync_copy(stripe_v, csum_sh.at[pl.ds(s * STRIPE, STRIPE)])
    plsc.subcore_barrier()

    base = (c * NS + s) * NB_A

    @pl.loop(0, NCH_A)
    def _(b):
        row0 = base + b * KA
        pltpu.sync_copy(ridx2_hbm.at[pl.ds(row0, KA)], ridx_c)
        pltpu.sync_copy(cidx2_hbm.at[pl.ds(row0, KA)], cidx_c)
        pltpu.sync_copy(ev2_hbm.at[pl.ds(row0, KA)], ev_c)
        hs = []
        for j in range(KA):
            hs.append(pltpu.async_copy(
                ev_c.at[j], rsum_sh.at[ridx_c.at[j]], ssem, add=True))
            hs.append(pltpu.async_copy(
                ev_c.at[j], csum_sh.at[cidx_c.at[j]], ssem, add=True))
        for hh in hs:
            hh.wait()

    plsc.subcore_barrier()
    pltpu.sync_copy(rsum_sh.at[pl.ds(s * STRIPE, STRIPE)], stripe_v)
    pltpu.sync_copy(stripe_v, sums_hbm.at[pl.ds((c * 2 + 0) * N2 + s * STRIPE, STRIPE)])
    pltpu.sync_copy(csum_sh.at[pl.ds(s * STRIPE, STRIPE)], stripe_v)
    pltpu.sync_copy(stripe_v, sums_hbm.at[pl.ds((c * 2 + 1) * N2 + s * STRIPE, STRIPE)])


@functools.partial(
    pl.kernel,
    out_type=jax.ShapeDtypeStruct((E2,), _f32),
    mesh=_mesh,
    scratch_types=[
        pltpu.VMEM((N2,), _f32),
        pltpu.VMEM((N2,), _f32),
        pltpu.VMEM((CW,), _i32),
        pltpu.VMEM((CW,), _i32),
        pltpu.VMEM((CW,), _f32),
        pltpu.VMEM((CW,), _f32),
    ],
    compiler_params=_sc_params,
)
def _weights_kernel(ridx_hbm, cidx_hbm, ev_hbm, rs_hbm, cs_hbm, nv_hbm,
                    rs_t, cs_t, ridx_v, cidx_v, ev_v, nv_v):
    c = lax.axis_index("c")
    s = lax.axis_index("s")
    pltpu.sync_copy(rs_hbm, rs_t)
    pltpu.sync_copy(cs_hbm, cs_t)
    base = (c * NS + s) * EPT_A

    @pl.loop(0, NCH_W)
    def _(b):
        off = base + b * CW
        pltpu.sync_copy(ridx_hbm.at[pl.ds(off, CW)], ridx_v)
        pltpu.sync_copy(cidx_hbm.at[pl.ds(off, CW)], cidx_v)
        pltpu.sync_copy(ev_hbm.at[pl.ds(off, CW)], ev_v)

        @pl.loop(0, CW // 16)
        def _(g):
            sl = pl.ds(g * 16, 16)
            nv_v[sl] = (ev_v[sl]
                        * plsc.load_gather(cs_t, [cidx_v[sl]])
                        * plsc.load_gather(rs_t, [ridx_v[sl]]))

        pltpu.sync_copy(nv_v, nv_hbm.at[pl.ds(off, CW)])


@functools.partial(
    pl.kernel,
    out_type=jax.ShapeDtypeStruct((2 * N2, H), _f32),
    mesh=_mesh,
    scratch_types=[
        pltpu.VMEM((K, BATCH), _i32),
        pltpu.VMEM((K, BATCH), _i32),
        pltpu.VMEM((K, BATCH), _f32),
        pltpu.VMEM((BATCH, H), _f32),
        pltpu.VMEM((BATCH, H), _f32),
        pltpu.VMEM((PR, H), _f32),
        pltpu.VMEM_SHARED((N2, H), _f32),
        pltpu.SemaphoreType.DMA,
        pltpu.SemaphoreType.DMA,
        pltpu.SemaphoreType.DMA,
        pltpu.SemaphoreType.DMA,
    ],
    compiler_params=_sc_params,
)
def _layer_kernel(curg_hbm, tidx2_hbm, ridx2_hbm, nv2_hbm,
                  out_hbm, tidx_c, ridx_c, nv_c,
                  rows0, rows1, sbuf, acc_sh,
                  gsem0, gsem1, ssem0, ssem1):
    h = lax.axis_index("c")
    s = lax.axis_index("s")
    rbuf = (rows0, rows1)
    gsem = (gsem0, gsem1)
    ssem = (ssem0, ssem1)

    @pl.loop(0, PR)
    def _(r):
        sbuf[r, pl.ds(0, 16)] = jnp.zeros((16,), _f32)
        sbuf[r, pl.ds(16, 16)] = jnp.zeros((16,), _f32)

    @pl.loop(0, STRIPE // PR)
    def _(i):
        pltpu.sync_copy(sbuf, acc_sh.at[pl.ds(s * STRIPE + i * PR, PR)])

    plsc.subcore_barrier()

    @pl.loop(0, NCH)
    def _(c):
        row0 = s * NB + c * K
        pltpu.sync_copy(tidx2_hbm.at[h, pl.ds(row0, K)], tidx_c)
        pltpu.sync_copy(ridx2_hbm.at[pl.ds(row0, K)], ridx_c)
        pltpu.sync_copy(nv2_hbm.at[pl.ds(row0, K)], nv_c)

        hg = [None] * K
        hs = [None] * K
        hg[0] = pltpu.async_copy(curg_hbm.at[tidx_c.at[0]], rbuf[0], gsem[0])
        for j in range(K):
            p = j & 1
            if j >= 1:
                hs[j - 1].wait()
            if j + 1 < K:
                hg[j + 1] = pltpu.async_copy(
                    curg_hbm.at[tidx_c.at[j + 1]], rbuf[1 - p], gsem[1 - p])
            hg[j].wait()

            @pl.loop(0, BATCH // 16)
            def _(g):
                for jj in range(16):
                    e = g * 16 + jj
                    w = plsc.load_gather(
                        nv_c, [jnp.zeros((16,), _i32) + j,
                               jnp.zeros((16,), _i32) + e])
                    rbuf[p][e, pl.ds(0, 16)] = rbuf[p][e, pl.ds(0, 16)] * w
                    rbuf[p][e, pl.ds(16, 16)] = rbuf[p][e, pl.ds(16, 16)] * w

            hs[j] = pltpu.async_copy(rbuf[p], acc_sh.at[ridx_c.at[j]],
                                     ssem[p], add=True)
        hs[K - 1].wait()

    plsc.subcore_barrier()

    @pl.loop(0, STRIPE // PR)
    def _(p):
        r0 = s * STRIPE + p * PR
        pltpu.sync_copy(acc_sh.at[pl.ds(r0, PR)], sbuf)

        @pl.loop(0, PR)
        def _(n):
            for half in (0, 16):
                x = sbuf[n, pl.ds(half, 16)]
                sbuf[n, pl.ds(half, 16)] = jnp.maximum(x, x * 0.01)

        pltpu.sync_copy(sbuf, out_hbm.at[pl.ds(h * N2 + r0, PR)])


def _norm_body(sums_ref, rs_ref, cs_ref):
    row = sums_ref[0] + sums_ref[2]
    col = sums_ref[1] + sums_ref[3]
    rs_ref[...] = 1.0 / (jnp.sqrt(row) + 1e-8)
    cs_ref[...] = 1.0 / (jnp.sqrt(col) + 1e-8)


_norm_call = pl.pallas_call(
    _norm_body,
    grid=(N2 // 128 // 8,),
    in_specs=[pl.BlockSpec((4, 8, 128), lambda i: (0, i, 0))],
    out_specs=[
        pl.BlockSpec((8, 128), lambda i: (i, 0)),
        pl.BlockSpec((8, 128), lambda i: (i, 0)),
    ],
    out_shape=[jax.ShapeDtypeStruct((N2 // 128, 128), _f32)] * 2,
)

_CB = 1568


def _combine_body(f_ref, c1_ref, c2_ref, o_ref):
    o_ref[...] = (f_ref[...] + c1_ref[...] + c2_ref[...]) * (1.0 / 3.0)


_combine_call = pl.pallas_call(
    _combine_body,
    grid=(2 * N2 // _CB,),
    in_specs=[pl.BlockSpec((_CB, H), lambda i: (i, 0))] * 3,
    out_specs=pl.BlockSpec((_CB, H), lambda i: (i, 0)),
    out_shape=jax.ShapeDtypeStruct((2 * N2, H), _f32),
)


@jax.jit
def _impl(users_feature, groups_feature, items_feature, edge_vals, edge_index):
    pad = E2 - E
    row = edge_index[0].astype(_i32)
    col = edge_index[1].astype(_i32)
    extra = (jnp.arange(pad, dtype=_i32) * 37) % N
    ridx_p = jnp.concatenate([row, extra])
    cidx_p = jnp.concatenate([col, extra])
    ev_p = jnp.concatenate([edge_vals, jnp.zeros((pad,), _f32)])

    ridx2 = ridx_p.reshape(NBT, BATCH)
    cidx2 = cidx_p.reshape(NBT, BATCH)
    ev2 = ev_p.reshape(NBT, BATCH)
    sums4 = _sums_kernel(ridx2, cidx2, ev2)
    rs2d, cs2d = _norm_call(sums4.reshape(4, N2 // 128, 128))
    rs = rs2d.reshape(N2)
    cs = cs2d.reshape(N2)

    nv = _weights_kernel(ridx_p, cidx_p, ev_p, rs, cs)

    tidx2 = jnp.stack([cidx_p, cidx_p + N2]).reshape(2, NBT, BATCH)
    nv2 = nv.reshape(NBT, BATCH)

    feats = jnp.concatenate([users_feature, groups_feature, items_feature], axis=0)
    feats_p = jnp.pad(feats, ((0, N2 - N), (0, 0)))
    fg = feats_p.reshape(N2, 2, H).transpose(1, 0, 2).reshape(2 * N2, H)

    def _step(cur, _):
        nxt = _layer_kernel(cur, tidx2, ridx2, nv2)
        return nxt, nxt

    _, curs = lax.scan(_step, fg, None, length=2)
    cur1, cur2 = curs[0], curs[1]

    outf = _combine_call(fg, cur1, cur2)
    outg = outf.reshape(2, N2, H)
    return jnp.concatenate([outg[0, :N], outg[1, :N]], axis=1)


def kernel(users_feature, groups_feature, items_feature, edge_vals, edge_index):
    return _impl(users_feature, groups_feature, items_feature, edge_vals,
                 edge_index)


# speedup vs baseline: 12.4725x; 1.1108x over previous
import functools

import jax
import jax.numpy as jnp
from jax import lax
from jax.experimental import pallas as pl
from jax.experimental.pallas import tpu as pltpu
from jax.experimental.pallas import tpu_sc as plsc

N = 50000
D = 64
H = 32
E = 800000
NC, NS = 2, 16
N2 = 50176
E2 = 802816
STRIPE = N2 // NS
BATCH = 128
EPT = E2 // NS
NB = EPT // BATCH
NBT = E2 // BATCH
EPT_A = E2 // (NC * NS)
NB_A = EPT_A // BATCH
KA = 7
NCH_A = NB_A // KA
CW = 14 * BATCH
NCH_W = EPT_A // CW
K = 8
NCH = NB // K
PR = 112
F = 2 * N2 * H

_mesh = plsc.VectorSubcoreMesh(
    core_axis_name="c", subcore_axis_name="s", num_cores=NC, num_subcores=NS
)

_f32 = jnp.float32
_i32 = jnp.int32

_sc_params = pltpu.CompilerParams(needs_layout_passes=False,
                                  use_tc_tiling_on_sc=False)


def _splat(vec_ref, i):
    return plsc.load_gather(vec_ref, [jnp.zeros((16,), _i32) + i])


@functools.partial(
    pl.kernel,
    out_type=jax.ShapeDtypeStruct((4 * N2,), _f32),
    mesh=_mesh,
    scratch_types=[
        pltpu.VMEM((KA, BATCH), _i32),
        pltpu.VMEM((KA, BATCH), _i32),
        pltpu.VMEM((KA, BATCH), _f32),
        pltpu.VMEM((STRIPE,), _f32),
        pltpu.VMEM_SHARED((N2,), _f32),
        pltpu.VMEM_SHARED((N2,), _f32),
        pltpu.SemaphoreType.DMA,
    ],
    compiler_params=_sc_params,
)
def _sums_kernel(ridx2_hbm, cidx2_hbm, ev2_hbm, sums_hbm,
                 ridx_c, cidx_c, ev_c, stripe_v, rsum_sh, csum_sh, ssem):
    c = lax.axis_index("c")
    s = lax.axis_index("s")

    @pl.loop(0, STRIPE // 16)
    def _(i):
        stripe_v[pl.ds(i * 16, 16)] = jnp.zeros((16,), _f32)

    pltpu.sync_copy(stripe_v, rsum_sh.at[pl.ds(s * STRIPE, STRIPE)])
    pltpu.sync_copy(stripe_v, csum_sh.at[pl.ds(s * STRIPE, STRIPE)])
    plsc.subcore_barrier()

    base = (c * NS + s) * NB_A

    @pl.loop(0, NCH_A)
    def _(b):
        row0 = base + b * KA
        pltpu.sync_copy(ridx2_hbm.at[pl.ds(row0, KA)], ridx_c)
        pltpu.sync_copy(cidx2_hbm.at[pl.ds(row0, KA)], cidx_c)
        pltpu.sync_copy(ev2_hbm.at[pl.ds(row0, KA)], ev_c)
        hs = []
        for j in range(KA):
            hs.append(pltpu.async_copy(
                ev_c.at[j], rsum_sh.at[ridx_c.at[j]], ssem, add=True))
            hs.append(pltpu.async_copy(
                ev_c.at[j], csum_sh.at[cidx_c.at[j]], ssem, add=True))
        for hh in hs:
            hh.wait()

    plsc.subcore_barrier()
    pltpu.sync_copy(rsum_sh.at[pl.ds(s * STRIPE, STRIPE)], stripe_v)
    pltpu.sync_copy(stripe_v, sums_hbm.at[pl.ds((c * 2 + 0) * N2 + s * STRIPE, STRIPE)])
    pltpu.sync_copy(csum_sh.at[pl.ds(s * STRIPE, STRIPE)], stripe_v)
    pltpu.sync_copy(stripe_v, sums_hbm.at[pl.ds((c * 2 + 1) * N2 + s * STRIPE, STRIPE)])


@functools.partial(
    pl.kernel,
    out_type=jax.ShapeDtypeStruct((E2,), _f32),
    mesh=_mesh,
    scratch_types=[
        pltpu.VMEM((N2,), _f32),
        pltpu.VMEM((N2,), _f32),
        pltpu.VMEM((CW,), _i32),
        pltpu.VMEM((CW,), _i32),
        pltpu.VMEM((CW,), _f32),
        pltpu.VMEM((CW,), _f32),
    ],
    compiler_params=_sc_params,
)
def _weights_kernel(ridx_hbm, cidx_hbm, ev_hbm, rs_hbm, cs_hbm, nv_hbm,
                    rs_t, cs_t, ridx_v, cidx_v, ev_v, nv_v):
    c = lax.axis_index("c")
    s = lax.axis_index("s")
    pltpu.sync_copy(rs_hbm, rs_t)
    pltpu.sync_copy(cs_hbm, cs_t)
    base = (c * NS + s) * EPT_A

    @pl.loop(0, NCH_W)
    def _(b):
        off = base + b * CW
        pltpu.sync_copy(ridx_hbm.at[pl.ds(off, CW)], ridx_v)
        pltpu.sync_copy(cidx_hbm.at[pl.ds(off, CW)], cidx_v)
        pltpu.sync_copy(ev_hbm.at[pl.ds(off, CW)], ev_v)

        @pl.loop(0, CW // 16)
        def _(g):
            sl = pl.ds(g * 16, 16)
            nv_v[sl] = (ev_v[sl]
                        * plsc.load_gather(cs_t, [cidx_v[sl]])
                        * plsc.load_gather(rs_t, [ridx_v[sl]]))

        pltpu.sync_copy(nv_v, nv_hbm.at[pl.ds(off, CW)])


@functools.partial(
    pl.kernel,
    out_type=[
        jax.ShapeDtypeStruct((2 * N2, H), _f32),
        jax.ShapeDtypeStruct((N2, 2, H), _f32),
    ],
    mesh=_mesh,
    scratch_types=[
        pltpu.VMEM((K, BATCH), _i32),
        pltpu.VMEM((K, BATCH), _i32),
        pltpu.VMEM((K, BATCH), _f32),
        pltpu.VMEM((BATCH, H), _f32),
        pltpu.VMEM((BATCH, H), _f32),
        pltpu.VMEM((PR, H), _f32),
        pltpu.VMEM((PR, H), _f32),
        pltpu.VMEM((PR, H), _f32),
        pltpu.VMEM_SHARED((N2, H), _f32),
        pltpu.SemaphoreType.DMA,
        pltpu.SemaphoreType.DMA,
        pltpu.SemaphoreType.DMA,
        pltpu.SemaphoreType.DMA,
    ],
    compiler_params=_sc_params,
)
def _layers_kernel(fg2_hbm, fgn_hbm, tidx4_hbm, ridx2_hbm, nv2_hbm,
                   cur1_hbm, final_hbm, tidx_c, ridx_c, nv_c,
                   rows0, rows1, sbuf, fbuf, c1buf, acc_sh,
                   gsem0, gsem1, ssem0, ssem1):
    h = lax.axis_index("c")
    s = lax.axis_index("s")
    rbuf = (rows0, rows1)
    gsem = (gsem0, gsem1)
    ssem = (ssem0, ssem1)

    def zero_acc():
        @pl.loop(0, PR)
        def _(r):
            sbuf[r, pl.ds(0, 16)] = jnp.zeros((16,), _f32)
            sbuf[r, pl.ds(16, 16)] = jnp.zeros((16,), _f32)

        @pl.loop(0, STRIPE // PR)
        def _(i):
            pltpu.sync_copy(sbuf, acc_sh.at[pl.ds(s * STRIPE + i * PR, PR)])

    def edge_phase(ti, src_hbm):
        @pl.loop(0, NCH)
        def _(c):
            row0 = s * NB + c * K
            pltpu.sync_copy(tidx4_hbm.at[ti, pl.ds(row0, K)], tidx_c)
            pltpu.sync_copy(ridx2_hbm.at[pl.ds(row0, K)], ridx_c)
            pltpu.sync_copy(nv2_hbm.at[pl.ds(row0, K)], nv_c)

            hg = [None] * K
            hs = [None] * K
            hg[0] = pltpu.async_copy(src_hbm.at[tidx_c.at[0]], rbuf[0],
                                     gsem[0])
            for j in range(K):
                p = j & 1
                if j >= 1:
                    hs[j - 1].wait()
                if j + 1 < K:
                    hg[j + 1] = pltpu.async_copy(
                        src_hbm.at[tidx_c.at[j + 1]], rbuf[1 - p],
                        gsem[1 - p])
                hg[j].wait()

                @pl.loop(0, BATCH // 16)
                def _(g):
                    for jj in range(16):
                        e = g * 16 + jj
                        w = plsc.load_gather(
                            nv_c, [jnp.zeros((16,), _i32) + j,
                                   jnp.zeros((16,), _i32) + e])
                        rbuf[p][e, pl.ds(0, 16)] = rbuf[p][e, pl.ds(0, 16)] * w
                        rbuf[p][e, pl.ds(16, 16)] = (
                            rbuf[p][e, pl.ds(16, 16)] * w)

                hs[j] = pltpu.async_copy(rbuf[p], acc_sh.at[ridx_c.at[j]],
                                         ssem[p], add=True)
            hs[K - 1].wait()

    zero_acc()
    plsc.subcore_barrier()
    edge_phase(h, fg2_hbm)
    plsc.subcore_barrier()

    @pl.loop(0, STRIPE // PR)
    def _(p):
        r0 = s * STRIPE + p * PR
        pltpu.sync_copy(acc_sh.at[pl.ds(r0, PR)], sbuf)

        @pl.loop(0, PR)
        def _(n):
            for half in (0, 16):
                x = sbuf[n, pl.ds(half, 16)]
                sbuf[n, pl.ds(half, 16)] = jnp.maximum(x, x * 0.01)

        pltpu.sync_copy(sbuf, cur1_hbm.at[pl.ds(h * N2 + r0, PR)])

    zero_acc()
    plsc.subcore_barrier()
    edge_phase(2 + h, cur1_hbm)
    plsc.subcore_barrier()

    @pl.loop(0, STRIPE // PR)
    def _(p):
        r0 = s * STRIPE + p * PR
        pltpu.sync_copy(acc_sh.at[pl.ds(r0, PR)], sbuf)
        pltpu.sync_copy(fgn_hbm.at[pl.ds(r0, PR), h], fbuf)
        pltpu.sync_copy(cur1_hbm.at[pl.ds(h * N2 + r0, PR)], c1buf)

        @pl.loop(0, PR)
        def _(n):
            for half in (0, 16):
                x = sbuf[n, pl.ds(half, 16)]
                x = jnp.maximum(x, x * 0.01)
                sbuf[n, pl.ds(half, 16)] = (
                    fbuf[n, pl.ds(half, 16)] + c1buf[n, pl.ds(half, 16)] + x
                ) * (1.0 / 3.0)

        pltpu.sync_copy(sbuf, final_hbm.at[pl.ds(r0, PR), h])


def _norm_body(sums_ref, rs_ref, cs_ref):
    row = sums_ref[0] + sums_ref[2]
    col = sums_ref[1] + sums_ref[3]
    rs_ref[...] = 1.0 / (jnp.sqrt(row) + 1e-8)
    cs_ref[...] = 1.0 / (jnp.sqrt(col) + 1e-8)


_norm_call = pl.pallas_call(
    _norm_body,
    grid=(N2 // 128 // 8,),
    in_specs=[pl.BlockSpec((4, 8, 128), lambda i: (0, i, 0))],
    out_specs=[
        pl.BlockSpec((8, 128), lambda i: (i, 0)),
        pl.BlockSpec((8, 128), lambda i: (i, 0)),
    ],
    out_shape=[jax.ShapeDtypeStruct((N2 // 128, 128), _f32)] * 2,
)

@jax.jit
def _impl(users_feature, groups_feature, items_feature, edge_vals, edge_index):
    pad = E2 - E
    row = edge_index[0].astype(_i32)
    col = edge_index[1].astype(_i32)
    extra = (jnp.arange(pad, dtype=_i32) * 37) % N
    ridx_p = jnp.concatenate([row, extra])
    cidx_p = jnp.concatenate([col, extra])
    ev_p = jnp.concatenate([edge_vals, jnp.zeros((pad,), _f32)])

    ridx2 = ridx_p.reshape(NBT, BATCH)
    cidx2 = cidx_p.reshape(NBT, BATCH)
    ev2 = ev_p.reshape(NBT, BATCH)
    sums4 = _sums_kernel(ridx2, cidx2, ev2)
    rs2d, cs2d = _norm_call(sums4.reshape(4, N2 // 128, 128))
    rs = rs2d.reshape(N2)
    cs = cs2d.reshape(N2)

    nv = _weights_kernel(ridx_p, cidx_p, ev_p, rs, cs)

    tidx4 = jnp.stack([2 * cidx_p, 2 * cidx_p + 1,
                       cidx_p, cidx_p + N2]).reshape(4, NBT, BATCH)
    nv2 = nv.reshape(NBT, BATCH)

    feats = jnp.concatenate([users_feature, groups_feature, items_feature], axis=0)
    feats_p = jnp.pad(feats, ((0, N2 - N), (0, 0)))
    fg2 = feats_p.reshape(2 * N2, H)
    fgn = feats_p.reshape(N2, 2, H)

    _, final3 = _layers_kernel(fg2, fgn, tidx4, ridx2, nv2)
    return final3.reshape(N2, D)[:N]


def kernel(users_feature, groups_feature, items_feature, edge_vals, edge_index):
    return _impl(users_feature, groups_feature, items_feature, edge_vals,
                 edge_index)


# speedup vs baseline: 13.9450x; 1.1181x over previous
import functools

import jax
import jax.numpy as jnp
from jax import lax
from jax.experimental import pallas as pl
from jax.experimental.pallas import tpu as pltpu
from jax.experimental.pallas import tpu_sc as plsc

N = 50000
D = 64
H = 32
E = 800000
NC, NS = 2, 16
N2 = 50176
E2 = 802816
STRIPE = N2 // NS
BATCH = 128
EPT = E2 // NS
NB = EPT // BATCH
NBT = E2 // BATCH
EPT_A = E2 // (NC * NS)
NB_A = EPT_A // BATCH
KA = 7
NCH_A = NB_A // KA
CW = 14 * BATCH
NCH_W = EPT_A // CW
K = 8
NCH = NB // K
PR = 112
F = 2 * N2 * H

_mesh = plsc.VectorSubcoreMesh(
    core_axis_name="c", subcore_axis_name="s", num_cores=NC, num_subcores=NS
)

_f32 = jnp.float32
_i32 = jnp.int32

_sc_params = pltpu.CompilerParams(needs_layout_passes=False,
                                  use_tc_tiling_on_sc=False)


def _splat(vec_ref, i):
    return plsc.load_gather(vec_ref, [jnp.zeros((16,), _i32) + i])


@functools.partial(
    pl.kernel,
    out_type=jax.ShapeDtypeStruct((4 * N2,), _f32),
    mesh=_mesh,
    scratch_types=[
        pltpu.VMEM((KA, BATCH), _i32),
        pltpu.VMEM((KA, BATCH), _i32),
        pltpu.VMEM((KA, BATCH), _f32),
        pltpu.VMEM((STRIPE,), _f32),
        pltpu.VMEM_SHARED((N2,), _f32),
        pltpu.VMEM_SHARED((N2,), _f32),
        pltpu.SemaphoreType.DMA,
    ],
    compiler_params=_sc_params,
)
def _sums_kernel(ridx2_hbm, cidx2_hbm, ev2_hbm, sums_hbm,
                 ridx_c, cidx_c, ev_c, stripe_v, rsum_sh, csum_sh, ssem):
    c = lax.axis_index("c")
    s = lax.axis_index("s")

    @pl.loop(0, STRIPE // 16)
    def _(i):
        stripe_v[pl.ds(i * 16, 16)] = jnp.zeros((16,), _f32)

    pltpu.sync_copy(stripe_v, rsum_sh.at[pl.ds(s * STRIPE, STRIPE)])
    pltpu.sync_copy(stripe_v, csum_sh.at[pl.ds(s * STRIPE, STRIPE)])
    plsc.subcore_barrier()

    base = (c * NS + s) * NB_A

    @pl.loop(0, NCH_A)
    def _(b):
        row0 = base + b * KA
        pltpu.sync_copy(ridx2_hbm.at[pl.ds(row0, KA)], ridx_c)
        pltpu.sync_copy(cidx2_hbm.at[pl.ds(row0, KA)], cidx_c)
        pltpu.sync_copy(ev2_hbm.at[pl.ds(row0, KA)], ev_c)
        hs = []
        for j in range(KA):
            hs.append(pltpu.async_copy(
                ev_c.at[j], rsum_sh.at[ridx_c.at[j]], ssem, add=True))
            hs.append(pltpu.async_copy(
                ev_c.at[j], csum_sh.at[cidx_c.at[j]], ssem, add=True))
        for hh in hs:
            hh.wait()

    plsc.subcore_barrier()
    pltpu.sync_copy(rsum_sh.at[pl.ds(s * STRIPE, STRIPE)], stripe_v)
    pltpu.sync_copy(stripe_v, sums_hbm.at[pl.ds((c * 2 + 0) * N2 + s * STRIPE, STRIPE)])
    pltpu.sync_copy(csum_sh.at[pl.ds(s * STRIPE, STRIPE)], stripe_v)
    pltpu.sync_copy(stripe_v, sums_hbm.at[pl.ds((c * 2 + 1) * N2 + s * STRIPE, STRIPE)])


@functools.partial(
    pl.kernel,
    out_type=jax.ShapeDtypeStruct((E2,), _f32),
    mesh=_mesh,
    scratch_types=[
        pltpu.VMEM((N2,), _f32),
        pltpu.VMEM((N2,), _f32),
        pltpu.VMEM((CW,), _i32),
        pltpu.VMEM((CW,), _i32),
        pltpu.VMEM((CW,), _f32),
        pltpu.VMEM((CW,), _f32),
    ],
    compiler_params=_sc_params,
)
def _weights_kernel(ridx_hbm, cidx_hbm, ev_hbm, rs_hbm, cs_hbm, nv_hbm,
                    rs_t, cs_t, ridx_v, cidx_v, ev_v, nv_v):
    c = lax.axis_index("c")
    s = lax.axis_index("s")
    pltpu.sync_copy(rs_hbm, rs_t)
    pltpu.sync_copy(cs_hbm, cs_t)
    base = (c * NS + s) * EPT_A

    @pl.loop(0, NCH_W)
    def _(b):
        off = base + b * CW
        pltpu.sync_copy(ridx_hbm.at[pl.ds(off, CW)], ridx_v)
        pltpu.sync_copy(cidx_hbm.at[pl.ds(off, CW)], cidx_v)
        pltpu.sync_copy(ev_hbm.at[pl.ds(off, CW)], ev_v)

        @pl.loop(0, CW // 16)
        def _(g):
            sl = pl.ds(g * 16, 16)
            nv_v[sl] = (ev_v[sl]
                        * plsc.load_gather(cs_t, [cidx_v[sl]])
                        * plsc.load_gather(rs_t, [ridx_v[sl]]))

        pltpu.sync_copy(nv_v, nv_hbm.at[pl.ds(off, CW)])


@functools.partial(
    pl.kernel,
    out_type=[
        jax.ShapeDtypeStruct((2 * N2, H), _f32),
        jax.ShapeDtypeStruct((N2, 2, H), _f32),
    ],
    mesh=_mesh,
    scratch_types=[
        pltpu.VMEM((2, K, BATCH), _i32),
        pltpu.VMEM((2, K, BATCH), _i32),
        pltpu.VMEM((2, K, BATCH), _f32),
        pltpu.VMEM((BATCH, H), _f32),
        pltpu.VMEM((BATCH, H), _f32),
        pltpu.VMEM((PR, H), _f32),
        pltpu.VMEM((PR, H), _f32),
        pltpu.VMEM((PR, H), _f32),
        pltpu.VMEM_SHARED((N2, H), _f32),
        pltpu.SemaphoreType.DMA,
        pltpu.SemaphoreType.DMA,
        pltpu.SemaphoreType.DMA,
        pltpu.SemaphoreType.DMA,
        pltpu.SemaphoreType.DMA,
    ],
    compiler_params=_sc_params,
)
def _layers_kernel(fg2_hbm, fgn_hbm, tidx4_hbm, ridx2_hbm, nv2_hbm,
                   cur1_hbm, final_hbm, tidx_c, ridx_c, nv_c,
                   rows0, rows1, sbuf, fbuf, c1buf, acc_sh,
                   gsem0, gsem1, ssem0, ssem1, isem):
    h = lax.axis_index("c")
    s = lax.axis_index("s")
    rbuf = (rows0, rows1)
    gsem = (gsem0, gsem1)
    ssem = (ssem0, ssem1)

    def zero_acc():
        @pl.loop(0, PR)
        def _(r):
            sbuf[r, pl.ds(0, 16)] = jnp.zeros((16,), _f32)
            sbuf[r, pl.ds(16, 16)] = jnp.zeros((16,), _f32)

        @pl.loop(0, STRIPE // PR)
        def _(i):
            pltpu.sync_copy(sbuf, acc_sh.at[pl.ds(s * STRIPE + i * PR, PR)])

    def edge_phase(ti, src_hbm):
        def drain_scatter(p):
            pltpu.make_async_copy(src_hbm.at[pl.ds(0, BATCH)], rbuf[p],
                                  ssem[p]).wait()

        base = s * NB
        pltpu.sync_copy(tidx4_hbm.at[ti, pl.ds(base, K)], tidx_c.at[0])
        pltpu.sync_copy(ridx2_hbm.at[pl.ds(base, K)], ridx_c.at[0])
        pltpu.sync_copy(nv2_hbm.at[pl.ds(base, K)], nv_c.at[0])

        @pl.loop(0, NCH)
        def _(c):
            q = c % 2
            row1 = base + (c + 1) * K

            hg = [None] * K

            def issue_gather(j):
                hg[j] = pltpu.async_copy(
                    src_hbm.at[tidx_c.at[q, j]], rbuf[j & 1], gsem[j & 1])

            for j in (0, 1):
                @pl.when(c > 0)
                def _(j=j):
                    drain_scatter(j)
                issue_gather(j)

            @pl.when(c + 1 < NCH)
            def _():
                pltpu.async_copy(tidx4_hbm.at[ti, pl.ds(row1, K)],
                                 tidx_c.at[1 - q], isem)
                pltpu.async_copy(ridx2_hbm.at[pl.ds(row1, K)],
                                 ridx_c.at[1 - q], isem)
                pltpu.async_copy(nv2_hbm.at[pl.ds(row1, K)],
                                 nv_c.at[1 - q], isem)

            for j in range(K):
                p = j & 1
                hg[j].wait()

                @pl.loop(0, BATCH // 16)
                def _(g):
                    for jj in range(16):
                        e = g * 16 + jj
                        w = plsc.load_gather(
                            nv_c, [jnp.zeros((16,), _i32) + q,
                                   jnp.zeros((16,), _i32) + j,
                                   jnp.zeros((16,), _i32) + e])
                        rbuf[p][e, pl.ds(0, 16)] = rbuf[p][e, pl.ds(0, 16)] * w
                        rbuf[p][e, pl.ds(16, 16)] = (
                            rbuf[p][e, pl.ds(16, 16)] * w)

                pltpu.async_copy(rbuf[p], acc_sh.at[ridx_c.at[q, j]],
                                 ssem[p], add=True)
                if j + 2 < K:
                    drain_scatter(p)
                    issue_gather(j + 2)

            @pl.when(c + 1 < NCH)
            def _():
                pltpu.make_async_copy(tidx4_hbm.at[ti, pl.ds(base, K)],
                                      tidx_c.at[1 - q], isem).wait()
                pltpu.make_async_copy(ridx2_hbm.at[pl.ds(base, K)],
                                      ridx_c.at[1 - q], isem).wait()
                pltpu.make_async_copy(nv2_hbm.at[pl.ds(base, K)],
                                      nv_c.at[1 - q], isem).wait()

        drain_scatter(0)
        drain_scatter(1)

    zero_acc()
    plsc.subcore_barrier()
    edge_phase(h, fg2_hbm)
    plsc.subcore_barrier()

    @pl.loop(0, STRIPE // PR)
    def _(p):
        r0 = s * STRIPE + p * PR
        pltpu.sync_copy(acc_sh.at[pl.ds(r0, PR)], sbuf)

        @pl.loop(0, PR)
        def _(n):
            for half in (0, 16):
                x = sbuf[n, pl.ds(half, 16)]
                sbuf[n, pl.ds(half, 16)] = jnp.maximum(x, x * 0.01)

        pltpu.sync_copy(sbuf, cur1_hbm.at[pl.ds(h * N2 + r0, PR)])

    zero_acc()
    plsc.subcore_barrier()
    edge_phase(2 + h, cur1_hbm)
    plsc.subcore_barrier()

    @pl.loop(0, STRIPE // PR)
    def _(p):
        r0 = s * STRIPE + p * PR
        pltpu.sync_copy(acc_sh.at[pl.ds(r0, PR)], sbuf)
        pltpu.sync_copy(fgn_hbm.at[pl.ds(r0, PR), h], fbuf)
        pltpu.sync_copy(cur1_hbm.at[pl.ds(h * N2 + r0, PR)], c1buf)

        @pl.loop(0, PR)
        def _(n):
            for half in (0, 16):
                x = sbuf[n, pl.ds(half, 16)]
                x = jnp.maximum(x, x * 0.01)
                sbuf[n, pl.ds(half, 16)] = (
                    fbuf[n, pl.ds(half, 16)] + c1buf[n, pl.ds(half, 16)] + x
                ) * (1.0 / 3.0)

        pltpu.sync_copy(sbuf, final_hbm.at[pl.ds(r0, PR), h])


def _norm_body(sums_ref, rs_ref, cs_ref):
    row = sums_ref[0] + sums_ref[2]
    col = sums_ref[1] + sums_ref[3]
    rs_ref[...] = 1.0 / (jnp.sqrt(row) + 1e-8)
    cs_ref[...] = 1.0 / (jnp.sqrt(col) + 1e-8)


_norm_call = pl.pallas_call(
    _norm_body,
    grid=(N2 // 128 // 8,),
    in_specs=[pl.BlockSpec((4, 8, 128), lambda i: (0, i, 0))],
    out_specs=[
        pl.BlockSpec((8, 128), lambda i: (i, 0)),
        pl.BlockSpec((8, 128), lambda i: (i, 0)),
    ],
    out_shape=[jax.ShapeDtypeStruct((N2 // 128, 128), _f32)] * 2,
)

@jax.jit
def _impl(users_feature, groups_feature, items_feature, edge_vals, edge_index):
    pad = E2 - E
    row = edge_index[0].astype(_i32)
    col = edge_index[1].astype(_i32)
    extra = (jnp.arange(pad, dtype=_i32) * 37) % N
    ridx_p = jnp.concatenate([row, extra])
    cidx_p = jnp.concatenate([col, extra])
    ev_p = jnp.concatenate([edge_vals, jnp.zeros((pad,), _f32)])

    ridx2 = ridx_p.reshape(NBT, BATCH)
    cidx2 = cidx_p.reshape(NBT, BATCH)
    ev2 = ev_p.reshape(NBT, BATCH)
    sums4 = _sums_kernel(ridx2, cidx2, ev2)
    rs2d, cs2d = _norm_call(sums4.reshape(4, N2 // 128, 128))
    rs = rs2d.reshape(N2)
    cs = cs2d.reshape(N2)

    nv = _weights_kernel(ridx_p, cidx_p, ev_p, rs, cs)

    tidx4 = jnp.stack([2 * cidx_p, 2 * cidx_p + 1,
                       cidx_p, cidx_p + N2]).reshape(4, NBT, BATCH)
    nv2 = nv.reshape(NBT, BATCH)

    feats = jnp.concatenate([users_feature, groups_feature, items_feature], axis=0)
    feats_p = jnp.pad(feats, ((0, N2 - N), (0, 0)))
    fg2 = feats_p.reshape(2 * N2, H)
    fgn = feats_p.reshape(N2, 2, H)

    _, final3 = _layers_kernel(fg2, fgn, tidx4, ridx2, nv2)
    return final3.reshape(N2, D)[:N]


def kernel(users_feature, groups_feature, items_feature, edge_vals, edge_index):
    return _impl(users_feature, groups_feature, items_feature, edge_vals,
                 edge_index)


# speedup vs baseline: 18.1593x; 1.3022x over previous
import functools

import jax
import jax.numpy as jnp
from jax import lax
from jax.experimental import pallas as pl
from jax.experimental.pallas import tpu as pltpu
from jax.experimental.pallas import tpu_sc as plsc

N = 50000
D = 64
H = 32
E = 800000
NC, NS = 2, 16
N2 = 50176
E2 = 802816
STRIPE = N2 // NS
BATCH = 128
EPT = E2 // NS
NB = EPT // BATCH
NBT = E2 // BATCH
EPT_A = E2 // (NC * NS)
NB_A = EPT_A // BATCH
KA = 7
NCH_A = NB_A // KA
CW = 14 * BATCH
NCH_W = EPT_A // CW
K = 8
NCH = NB // K
PR = 112
F = 2 * N2 * H

_mesh = plsc.VectorSubcoreMesh(
    core_axis_name="c", subcore_axis_name="s", num_cores=NC, num_subcores=NS
)

_f32 = jnp.float32
_i32 = jnp.int32

_sc_params = pltpu.CompilerParams(needs_layout_passes=False,
                                  use_tc_tiling_on_sc=False)


def _splat(vec_ref, i):
    return plsc.load_gather(vec_ref, [jnp.zeros((16,), _i32) + i])


@functools.partial(
    pl.kernel,
    out_type=jax.ShapeDtypeStruct((4 * N2,), _f32),
    mesh=_mesh,
    scratch_types=[
        pltpu.VMEM((KA, BATCH), _i32),
        pltpu.VMEM((KA, BATCH), _i32),
        pltpu.VMEM((KA, BATCH), _f32),
        pltpu.VMEM((STRIPE,), _f32),
        pltpu.VMEM_SHARED((N2,), _f32),
        pltpu.VMEM_SHARED((N2,), _f32),
        pltpu.SemaphoreType.DMA,
    ],
    compiler_params=_sc_params,
)
def _sums_kernel(ridx2_hbm, cidx2_hbm, ev2_hbm, sums_hbm,
                 ridx_c, cidx_c, ev_c, stripe_v, rsum_sh, csum_sh, ssem):
    c = lax.axis_index("c")
    s = lax.axis_index("s")

    @pl.loop(0, STRIPE // 16)
    def _(i):
        stripe_v[pl.ds(i * 16, 16)] = jnp.zeros((16,), _f32)

    pltpu.sync_copy(stripe_v, rsum_sh.at[pl.ds(s * STRIPE, STRIPE)])
    pltpu.sync_copy(stripe_v, csum_sh.at[pl.ds(s * STRIPE, STRIPE)])
    plsc.subcore_barrier()

    base = (c * NS + s) * NB_A

    @pl.loop(0, NCH_A)
    def _(b):
        row0 = base + b * KA
        pltpu.sync_copy(ridx2_hbm.at[pl.ds(row0, KA)], ridx_c)
        pltpu.sync_copy(cidx2_hbm.at[pl.ds(row0, KA)], cidx_c)
        pltpu.sync_copy(ev2_hbm.at[pl.ds(row0, KA)], ev_c)
        hs = []
        for j in range(KA):
            hs.append(pltpu.async_copy(
                ev_c.at[j], rsum_sh.at[ridx_c.at[j]], ssem, add=True))
            hs.append(pltpu.async_copy(
                ev_c.at[j], csum_sh.at[cidx_c.at[j]], ssem, add=True))
        for hh in hs:
            hh.wait()

    plsc.subcore_barrier()
    pltpu.sync_copy(rsum_sh.at[pl.ds(s * STRIPE, STRIPE)], stripe_v)
    pltpu.sync_copy(stripe_v, sums_hbm.at[pl.ds((c * 2 + 0) * N2 + s * STRIPE, STRIPE)])
    pltpu.sync_copy(csum_sh.at[pl.ds(s * STRIPE, STRIPE)], stripe_v)
    pltpu.sync_copy(stripe_v, sums_hbm.at[pl.ds((c * 2 + 1) * N2 + s * STRIPE, STRIPE)])


@functools.partial(
    pl.kernel,
    out_type=jax.ShapeDtypeStruct((E2,), _f32),
    mesh=_mesh,
    scratch_types=[
        pltpu.VMEM((N2,), _f32),
        pltpu.VMEM((N2,), _f32),
        pltpu.VMEM((CW,), _i32),
        pltpu.VMEM((CW,), _i32),
        pltpu.VMEM((CW,), _f32),
        pltpu.VMEM((CW,), _f32),
    ],
    compiler_params=_sc_params,
)
def _weights_kernel(ridx_hbm, cidx_hbm, ev_hbm, rs_hbm, cs_hbm, nv_hbm,
                    rs_t, cs_t, ridx_v, cidx_v, ev_v, nv_v):
    c = lax.axis_index("c")
    s = lax.axis_index("s")
    pltpu.sync_copy(rs_hbm, rs_t)
    pltpu.sync_copy(cs_hbm, cs_t)
    base = (c * NS + s) * EPT_A

    @pl.loop(0, NCH_W)
    def _(b):
        off = base + b * CW
        pltpu.sync_copy(ridx_hbm.at[pl.ds(off, CW)], ridx_v)
        pltpu.sync_copy(cidx_hbm.at[pl.ds(off, CW)], cidx_v)
        pltpu.sync_copy(ev_hbm.at[pl.ds(off, CW)], ev_v)

        @pl.loop(0, CW // 16)
        def _(g):
            sl = pl.ds(g * 16, 16)
            nv_v[sl] = (ev_v[sl]
                        * plsc.load_gather(cs_t, [cidx_v[sl]])
                        * plsc.load_gather(rs_t, [ridx_v[sl]]))

        pltpu.sync_copy(nv_v, nv_hbm.at[pl.ds(off, CW)])


@functools.partial(
    pl.kernel,
    out_type=[
        jax.ShapeDtypeStruct((2 * N2, H), _f32),
        jax.ShapeDtypeStruct((N2, 2, H), _f32),
    ],
    mesh=_mesh,
    scratch_types=[
        pltpu.VMEM((2, K, BATCH), _i32),
        pltpu.VMEM((2, K, BATCH), _i32),
        pltpu.VMEM((2, K, BATCH), _f32),
        pltpu.VMEM((BATCH, H), _f32),
        pltpu.VMEM((BATCH, H), _f32),
        pltpu.VMEM((PR, H), _f32),
        pltpu.VMEM((PR, H), _f32),
        pltpu.VMEM((PR, H), _f32),
        pltpu.VMEM_SHARED((N2, H), _f32),
        pltpu.SemaphoreType.DMA,
        pltpu.SemaphoreType.DMA,
        pltpu.SemaphoreType.DMA,
        pltpu.SemaphoreType.DMA,
        pltpu.SemaphoreType.DMA,
    ],
    compiler_params=_sc_params,
)
def _layers_kernel(fg2_hbm, fgn_hbm, tidx4_hbm, ridx2_hbm, nv2_hbm,
                   cur1_hbm, final_hbm, tidx_c, ridx_c, nv_c,
                   rows0, rows1, sbuf, fbuf, c1buf, acc_sh,
                   gsem0, gsem1, ssem0, ssem1, isem):
    h = lax.axis_index("c")
    s = lax.axis_index("s")
    rbuf = (rows0, rows1)
    gsem = (gsem0, gsem1)
    ssem = (ssem0, ssem1)

    def zero_acc():
        @pl.loop(0, PR)
        def _(r):
            sbuf[r, pl.ds(0, 16)] = jnp.zeros((16,), _f32)
            sbuf[r, pl.ds(16, 16)] = jnp.zeros((16,), _f32)

        @pl.loop(0, STRIPE // PR)
        def _(i):
            pltpu.sync_copy(sbuf, acc_sh.at[pl.ds(s * STRIPE + i * PR, PR)])

    def edge_phase(ti, src_hbm):
        def drain_scatter(p):
            pltpu.make_async_copy(src_hbm.at[pl.ds(0, BATCH)], rbuf[p],
                                  ssem[p]).wait()

        base = s * NB
        pltpu.sync_copy(tidx4_hbm.at[ti, pl.ds(base, K)], tidx_c.at[0])
        pltpu.sync_copy(ridx2_hbm.at[pl.ds(base, K)], ridx_c.at[0])
        pltpu.sync_copy(nv2_hbm.at[pl.ds(base, K)], nv_c.at[0])

        @pl.loop(0, NCH)
        def _(c):
            q = c % 2
            row1 = base + (c + 1) * K

            hg = [None] * K

            def issue_gather(j):
                hg[j] = pltpu.async_copy(
                    src_hbm.at[tidx_c.at[q, j]], rbuf[j & 1], gsem[j & 1])

            for j in (0, 1):
                @pl.when(c > 0)
                def _(j=j):
                    drain_scatter(j)
                issue_gather(j)

            @pl.when(c + 1 < NCH)
            def _():
                pltpu.async_copy(tidx4_hbm.at[ti, pl.ds(row1, K)],
                                 tidx_c.at[1 - q], isem)
                pltpu.async_copy(ridx2_hbm.at[pl.ds(row1, K)],
                                 ridx_c.at[1 - q], isem)
                pltpu.async_copy(nv2_hbm.at[pl.ds(row1, K)],
                                 nv_c.at[1 - q], isem)

            for j in range(K):
                p = j & 1
                hg[j].wait()

                @pl.loop(0, BATCH // 16)
                def _(g):
                    w16 = nv_c[q, j, pl.ds(g * 16, 16)]
                    for jj in range(16):
                        e = g * 16 + jj
                        w = w16.at[jnp.zeros((16,), _i32) + jj].get(
                            mode="promise_in_bounds")
                        rbuf[p][e, pl.ds(0, 16)] = rbuf[p][e, pl.ds(0, 16)] * w
                        rbuf[p][e, pl.ds(16, 16)] = (
                            rbuf[p][e, pl.ds(16, 16)] * w)

                pltpu.async_copy(rbuf[p], acc_sh.at[ridx_c.at[q, j]],
                                 ssem[p], add=True)
                if j + 2 < K:
                    drain_scatter(p)
                    issue_gather(j + 2)

            @pl.when(c + 1 < NCH)
            def _():
                pltpu.make_async_copy(tidx4_hbm.at[ti, pl.ds(base, K)],
                                      tidx_c.at[1 - q], isem).wait()
                pltpu.make_async_copy(ridx2_hbm.at[pl.ds(base, K)],
                                      ridx_c.at[1 - q], isem).wait()
                pltpu.make_async_copy(nv2_hbm.at[pl.ds(base, K)],
                                      nv_c.at[1 - q], isem).wait()

        drain_scatter(0)
        drain_scatter(1)

    zero_acc()
    plsc.subcore_barrier()
    edge_phase(h, fg2_hbm)
    plsc.subcore_barrier()

    @pl.loop(0, STRIPE // PR)
    def _(p):
        r0 = s * STRIPE + p * PR
        pltpu.sync_copy(acc_sh.at[pl.ds(r0, PR)], sbuf)

        @pl.loop(0, PR)
        def _(n):
            for half in (0, 16):
                x = sbuf[n, pl.ds(half, 16)]
                sbuf[n, pl.ds(half, 16)] = jnp.maximum(x, x * 0.01)

        pltpu.sync_copy(sbuf, cur1_hbm.at[pl.ds(h * N2 + r0, PR)])

    zero_acc()
    plsc.subcore_barrier()
    edge_phase(2 + h, cur1_hbm)
    plsc.subcore_barrier()

    @pl.loop(0, STRIPE // PR)
    def _(p):
        r0 = s * STRIPE + p * PR
        pltpu.sync_copy(acc_sh.at[pl.ds(r0, PR)], sbuf)
        pltpu.sync_copy(fgn_hbm.at[pl.ds(r0, PR), h], fbuf)
        pltpu.sync_copy(cur1_hbm.at[pl.ds(h * N2 + r0, PR)], c1buf)

        @pl.loop(0, PR)
        def _(n):
            for half in (0, 16):
                x = sbuf[n, pl.ds(half, 16)]
                x = jnp.maximum(x, x * 0.01)
                sbuf[n, pl.ds(half, 16)] = (
                    fbuf[n, pl.ds(half, 16)] + c1buf[n, pl.ds(half, 16)] + x
                ) * (1.0 / 3.0)

        pltpu.sync_copy(sbuf, final_hbm.at[pl.ds(r0, PR), h])


def _norm_body(sums_ref, rs_ref, cs_ref):
    row = sums_ref[0] + sums_ref[2]
    col = sums_ref[1] + sums_ref[3]
    rs_ref[...] = 1.0 / (jnp.sqrt(row) + 1e-8)
    cs_ref[...] = 1.0 / (jnp.sqrt(col) + 1e-8)


_norm_call = pl.pallas_call(
    _norm_body,
    grid=(N2 // 128 // 8,),
    in_specs=[pl.BlockSpec((4, 8, 128), lambda i: (0, i, 0))],
    out_specs=[
        pl.BlockSpec((8, 128), lambda i: (i, 0)),
        pl.BlockSpec((8, 128), lambda i: (i, 0)),
    ],
    out_shape=[jax.ShapeDtypeStruct((N2 // 128, 128), _f32)] * 2,
)

@jax.jit
def _impl(users_feature, groups_feature, items_feature, edge_vals, edge_index):
    pad = E2 - E
    row = edge_index[0].astype(_i32)
    col = edge_index[1].astype(_i32)
    extra = (jnp.arange(pad, dtype=_i32) * 37) % N
    ridx_p = jnp.concatenate([row, extra])
    cidx_p = jnp.concatenate([col, extra])
    ev_p = jnp.concatenate([edge_vals, jnp.zeros((pad,), _f32)])

    ridx2 = ridx_p.reshape(NBT, BATCH)
    cidx2 = cidx_p.reshape(NBT, BATCH)
    ev2 = ev_p.reshape(NBT, BATCH)
    sums4 = _sums_kernel(ridx2, cidx2, ev2)
    rs2d, cs2d = _norm_call(sums4.reshape(4, N2 // 128, 128))
    rs = rs2d.reshape(N2)
    cs = cs2d.reshape(N2)

    nv = _weights_kernel(ridx_p, cidx_p, ev_p, rs, cs)

    tidx4 = jnp.stack([2 * cidx_p, 2 * cidx_p + 1,
                       cidx_p, cidx_p + N2]).reshape(4, NBT, BATCH)
    nv2 = nv.reshape(NBT, BATCH)

    feats = jnp.concatenate([users_feature, groups_feature, items_feature], axis=0)
    feats_p = jnp.pad(feats, ((0, N2 - N), (0, 0)))
    fg2 = feats_p.reshape(2 * N2, H)
    fgn = feats_p.reshape(N2, 2, H)

    _, final3 = _layers_kernel(fg2, fgn, tidx4, ridx2, nv2)
    return final3.reshape(N2, D)[:N]


def kernel(users_feature, groups_feature, items_feature, edge_vals, edge_index):
    return _impl(users_feature, groups_feature, items_feature, edge_vals,
                 edge_index)


# speedup vs baseline: 18.6495x; 1.0270x over previous
import functools

import jax
import jax.numpy as jnp
from jax import lax
from jax.experimental import pallas as pl
from jax.experimental.pallas import tpu as pltpu
from jax.experimental.pallas import tpu_sc as plsc

N = 50000
D = 64
H = 32
E = 800000
NC, NS = 2, 16
N2 = 50176
E2 = 802816
STRIPE = N2 // NS
BATCH = 128
EPT = E2 // NS
NB = EPT // BATCH
NBT = E2 // BATCH
EPT_A = E2 // (NC * NS)
NB_A = EPT_A // BATCH
KA = 7
NCH_A = NB_A // KA
CW = 14 * BATCH
NCH_W = EPT_A // CW
K = 8
NCH = NB // K
PR = 56
F = 2 * N2 * H

_mesh = plsc.VectorSubcoreMesh(
    core_axis_name="c", subcore_axis_name="s", num_cores=NC, num_subcores=NS
)

_f32 = jnp.float32
_i32 = jnp.int32

_sc_params = pltpu.CompilerParams(needs_layout_passes=False,
                                  use_tc_tiling_on_sc=False)


def _splat(vec_ref, i):
    return plsc.load_gather(vec_ref, [jnp.zeros((16,), _i32) + i])


@functools.partial(
    pl.kernel,
    out_type=jax.ShapeDtypeStruct((4 * N2,), _f32),
    mesh=_mesh,
    scratch_types=[
        pltpu.VMEM((KA, BATCH), _i32),
        pltpu.VMEM((KA, BATCH), _i32),
        pltpu.VMEM((KA, BATCH), _f32),
        pltpu.VMEM((STRIPE,), _f32),
        pltpu.VMEM_SHARED((N2,), _f32),
        pltpu.VMEM_SHARED((N2,), _f32),
        pltpu.SemaphoreType.DMA,
    ],
    compiler_params=_sc_params,
)
def _sums_kernel(ridx2_hbm, cidx2_hbm, ev2_hbm, sums_hbm,
                 ridx_c, cidx_c, ev_c, stripe_v, rsum_sh, csum_sh, ssem):
    c = lax.axis_index("c")
    s = lax.axis_index("s")

    @pl.loop(0, STRIPE // 16)
    def _(i):
        stripe_v[pl.ds(i * 16, 16)] = jnp.zeros((16,), _f32)

    pltpu.sync_copy(stripe_v, rsum_sh.at[pl.ds(s * STRIPE, STRIPE)])
    pltpu.sync_copy(stripe_v, csum_sh.at[pl.ds(s * STRIPE, STRIPE)])
    plsc.subcore_barrier()

    base = (c * NS + s) * NB_A

    @pl.loop(0, NCH_A)
    def _(b):
        row0 = base + b * KA
        pltpu.sync_copy(ridx2_hbm.at[pl.ds(row0, KA)], ridx_c)
        pltpu.sync_copy(cidx2_hbm.at[pl.ds(row0, KA)], cidx_c)
        pltpu.sync_copy(ev2_hbm.at[pl.ds(row0, KA)], ev_c)
        hs = []
        for j in range(KA):
            hs.append(pltpu.async_copy(
                ev_c.at[j], rsum_sh.at[ridx_c.at[j]], ssem, add=True))
            hs.append(pltpu.async_copy(
                ev_c.at[j], csum_sh.at[cidx_c.at[j]], ssem, add=True))
        for hh in hs:
            hh.wait()

    plsc.subcore_barrier()
    pltpu.sync_copy(rsum_sh.at[pl.ds(s * STRIPE, STRIPE)], stripe_v)
    pltpu.sync_copy(stripe_v, sums_hbm.at[pl.ds((c * 2 + 0) * N2 + s * STRIPE, STRIPE)])
    pltpu.sync_copy(csum_sh.at[pl.ds(s * STRIPE, STRIPE)], stripe_v)
    pltpu.sync_copy(stripe_v, sums_hbm.at[pl.ds((c * 2 + 1) * N2 + s * STRIPE, STRIPE)])


@functools.partial(
    pl.kernel,
    out_type=jax.ShapeDtypeStruct((E2,), _f32),
    mesh=_mesh,
    scratch_types=[
        pltpu.VMEM((N2,), _f32),
        pltpu.VMEM((N2,), _f32),
        pltpu.VMEM((CW,), _i32),
        pltpu.VMEM((CW,), _i32),
        pltpu.VMEM((CW,), _f32),
        pltpu.VMEM((CW,), _f32),
    ],
    compiler_params=_sc_params,
)
def _weights_kernel(ridx_hbm, cidx_hbm, ev_hbm, rs_hbm, cs_hbm, nv_hbm,
                    rs_t, cs_t, ridx_v, cidx_v, ev_v, nv_v):
    c = lax.axis_index("c")
    s = lax.axis_index("s")
    pltpu.sync_copy(rs_hbm, rs_t)
    pltpu.sync_copy(cs_hbm, cs_t)
    base = (c * NS + s) * EPT_A

    @pl.loop(0, NCH_W)
    def _(b):
        off = base + b * CW
        pltpu.sync_copy(ridx_hbm.at[pl.ds(off, CW)], ridx_v)
        pltpu.sync_copy(cidx_hbm.at[pl.ds(off, CW)], cidx_v)
        pltpu.sync_copy(ev_hbm.at[pl.ds(off, CW)], ev_v)

        @pl.loop(0, CW // 16)
        def _(g):
            sl = pl.ds(g * 16, 16)
            nv_v[sl] = (ev_v[sl]
                        * plsc.load_gather(cs_t, [cidx_v[sl]])
                        * plsc.load_gather(rs_t, [ridx_v[sl]]))

        pltpu.sync_copy(nv_v, nv_hbm.at[pl.ds(off, CW)])


@functools.partial(
    pl.kernel,
    out_type=[
        jax.ShapeDtypeStruct((2 * N2, H), _f32),
        jax.ShapeDtypeStruct((N2, 2, H), _f32),
    ],
    mesh=_mesh,
    scratch_types=[
        pltpu.VMEM((2, K, BATCH), _i32),
        pltpu.VMEM((2, K, BATCH), _i32),
        pltpu.VMEM((2, K, BATCH), _f32),
        pltpu.VMEM((BATCH, H), _f32),
        pltpu.VMEM((BATCH, H), _f32),
        pltpu.VMEM((BATCH, H), _f32),
        pltpu.VMEM((BATCH, H), _f32),
        pltpu.VMEM((PR, H), _f32),
        pltpu.VMEM((PR, H), _f32),
        pltpu.VMEM((PR, H), _f32),
        pltpu.VMEM_SHARED((N2, H), _f32),
        pltpu.SemaphoreType.DMA,
        pltpu.SemaphoreType.DMA,
        pltpu.SemaphoreType.DMA,
        pltpu.SemaphoreType.DMA,
        pltpu.SemaphoreType.DMA,
        pltpu.SemaphoreType.DMA,
        pltpu.SemaphoreType.DMA,
        pltpu.SemaphoreType.DMA,
        pltpu.SemaphoreType.DMA,
    ],
    compiler_params=_sc_params,
)
def _layers_kernel(fg2_hbm, fgn_hbm, tidx4_hbm, ridx2_hbm, nv2_hbm,
                   cur1_hbm, final_hbm, tidx_c, ridx_c, nv_c,
                   rows0, rows1, rows2, rows3, sbuf, fbuf, c1buf, acc_sh,
                   gsem0, gsem1, gsem2, gsem3,
                   ssem0, ssem1, ssem2, ssem3, isem):
    h = lax.axis_index("c")
    s = lax.axis_index("s")
    rbuf = (rows0, rows1, rows2, rows3)
    gsem = (gsem0, gsem1, gsem2, gsem3)
    ssem = (ssem0, ssem1, ssem2, ssem3)

    def zero_acc():
        @pl.loop(0, PR)
        def _(r):
            sbuf[r, pl.ds(0, 16)] = jnp.zeros((16,), _f32)
            sbuf[r, pl.ds(16, 16)] = jnp.zeros((16,), _f32)

        @pl.loop(0, STRIPE // PR)
        def _(i):
            pltpu.sync_copy(sbuf, acc_sh.at[pl.ds(s * STRIPE + i * PR, PR)])

    def edge_phase(ti, src_hbm):
        def drain_scatter(p):
            pltpu.make_async_copy(src_hbm.at[pl.ds(0, BATCH)], rbuf[p],
                                  ssem[p]).wait()

        base = s * NB
        pltpu.sync_copy(tidx4_hbm.at[ti, pl.ds(base, K)], tidx_c.at[0])
        pltpu.sync_copy(ridx2_hbm.at[pl.ds(base, K)], ridx_c.at[0])
        pltpu.sync_copy(nv2_hbm.at[pl.ds(base, K)], nv_c.at[0])

        @pl.loop(0, NCH)
        def _(c):
            q = c % 2
            row1 = base + (c + 1) * K

            hg = [None] * K

            def issue_gather(j):
                hg[j] = pltpu.async_copy(
                    src_hbm.at[tidx_c.at[q, j]], rbuf[j & 3], gsem[j & 3])

            for b in (0, 1):
                @pl.when(c > 0)
                def _(b=b):
                    drain_scatter(b)
                issue_gather(b)

            for j in range(K):
                p = j & 3
                hg[j].wait()

                @pl.loop(0, BATCH // 16)
                def _(g):
                    w16 = nv_c[q, j, pl.ds(g * 16, 16)]
                    for jj in range(16):
                        e = g * 16 + jj
                        w = w16.at[jnp.zeros((16,), _i32) + jj].get(
                            mode="promise_in_bounds")
                        rbuf[p][e, pl.ds(0, 16)] = rbuf[p][e, pl.ds(0, 16)] * w
                        rbuf[p][e, pl.ds(16, 16)] = (
                            rbuf[p][e, pl.ds(16, 16)] * w)

                pltpu.async_copy(rbuf[p], acc_sh.at[ridx_c.at[q, j]],
                                 ssem[p], add=True)
                if j + 2 < K:
                    pb = (j + 2) & 3
                    if j >= 2:
                        drain_scatter(pb)
                    else:
                        @pl.when(c > 0)
                        def _(pb=pb):
                            drain_scatter(pb)
                    issue_gather(j + 2)
                if j == 1:
                    @pl.when(c + 1 < NCH)
                    def _():
                        pltpu.async_copy(tidx4_hbm.at[ti, pl.ds(row1, K)],
                                         tidx_c.at[1 - q], isem)
                        pltpu.async_copy(ridx2_hbm.at[pl.ds(row1, K)],
                                         ridx_c.at[1 - q], isem)
                        pltpu.async_copy(nv2_hbm.at[pl.ds(row1, K)],
                                         nv_c.at[1 - q], isem)

            @pl.when(c + 1 < NCH)
            def _():
                pltpu.make_async_copy(tidx4_hbm.at[ti, pl.ds(base, K)],
                                      tidx_c.at[1 - q], isem).wait()
                pltpu.make_async_copy(ridx2_hbm.at[pl.ds(base, K)],
                                      ridx_c.at[1 - q], isem).wait()
                pltpu.make_async_copy(nv2_hbm.at[pl.ds(base, K)],
                                      nv_c.at[1 - q], isem).wait()

        drain_scatter(0)
        drain_scatter(1)
        drain_scatter(2)
        drain_scatter(3)

    zero_acc()
    plsc.subcore_barrier()
    edge_phase(h, fg2_hbm)
    plsc.subcore_barrier()

    @pl.loop(0, STRIPE // PR)
    def _(p):
        r0 = s * STRIPE + p * PR
        pltpu.sync_copy(acc_sh.at[pl.ds(r0, PR)], sbuf)

        @pl.loop(0, PR)
        def _(n):
            for half in (0, 16):
                x = sbuf[n, pl.ds(half, 16)]
                sbuf[n, pl.ds(half, 16)] = jnp.maximum(x, x * 0.01)

        pltpu.sync_copy(sbuf, cur1_hbm.at[pl.ds(h * N2 + r0, PR)])

    zero_acc()
    plsc.subcore_barrier()
    edge_phase(2 + h, cur1_hbm)
    plsc.subcore_barrier()

    @pl.loop(0, STRIPE // PR)
    def _(p):
        r0 = s * STRIPE + p * PR
        pltpu.sync_copy(acc_sh.at[pl.ds(r0, PR)], sbuf)
        pltpu.sync_copy(fgn_hbm.at[pl.ds(r0, PR), h], fbuf)
        pltpu.sync_copy(cur1_hbm.at[pl.ds(h * N2 + r0, PR)], c1buf)

        @pl.loop(0, PR)
        def _(n):
            for half in (0, 16):
                x = sbuf[n, pl.ds(half, 16)]
                x = jnp.maximum(x, x * 0.01)
                sbuf[n, pl.ds(half, 16)] = (
                    fbuf[n, pl.ds(half, 16)] + c1buf[n, pl.ds(half, 16)] + x
                ) * (1.0 / 3.0)

        pltpu.sync_copy(sbuf, final_hbm.at[pl.ds(r0, PR), h])


def _norm_body(sums_ref, rs_ref, cs_ref):
    row = sums_ref[0] + sums_ref[2]
    col = sums_ref[1] + sums_ref[3]
    rs_ref[...] = 1.0 / (jnp.sqrt(row) + 1e-8)
    cs_ref[...] = 1.0 / (jnp.sqrt(col) + 1e-8)


_norm_call = pl.pallas_call(
    _norm_body,
    grid=(N2 // 128 // 8,),
    in_specs=[pl.BlockSpec((4, 8, 128), lambda i: (0, i, 0))],
    out_specs=[
        pl.BlockSpec((8, 128), lambda i: (i, 0)),
        pl.BlockSpec((8, 128), lambda i: (i, 0)),
    ],
    out_shape=[jax.ShapeDtypeStruct((N2 // 128, 128), _f32)] * 2,
)

@jax.jit
def _impl(users_feature, groups_feature, items_feature, edge_vals, edge_index):
    pad = E2 - E
    row = edge_index[0].astype(_i32)
    col = edge_index[1].astype(_i32)
    extra = (jnp.arange(pad, dtype=_i32) * 37) % N
    ridx_p = jnp.concatenate([row, extra])
    cidx_p = jnp.concatenate([col, extra])
    ev_p = jnp.concatenate([edge_vals, jnp.zeros((pad,), _f32)])

    ridx2 = ridx_p.reshape(NBT, BATCH)
    cidx2 = cidx_p.reshape(NBT, BATCH)
    ev2 = ev_p.reshape(NBT, BATCH)
    sums4 = _sums_kernel(ridx2, cidx2, ev2)
    rs2d, cs2d = _norm_call(sums4.reshape(4, N2 // 128, 128))
    rs = rs2d.reshape(N2)
    cs = cs2d.reshape(N2)

    nv = _weights_kernel(ridx_p, cidx_p, ev_p, rs, cs)

    tidx4 = jnp.stack([2 * cidx_p, 2 * cidx_p + 1,
                       cidx_p, cidx_p + N2]).reshape(4, NBT, BATCH)
    nv2 = nv.reshape(NBT, BATCH)

    feats = jnp.concatenate([users_feature, groups_feature, items_feature], axis=0)
    feats_p = jnp.pad(feats, ((0, N2 - N), (0, 0)))
    fg2 = feats_p.reshape(2 * N2, H)
    fgn = feats_p.reshape(N2, 2, H)

    _, final3 = _layers_kernel(fg2, fgn, tidx4, ridx2, nv2)
    return final3.reshape(N2, D)[:N]


def kernel(users_feature, groups_feature, items_feature, edge_vals, edge_index):
    return _impl(users_feature, groups_feature, items_feature, edge_vals,
                 edge_index)
